# Initial kernel scaffold; baseline (speedup 1.0000x reference)
import jax
import jax.numpy as jnp
from jax.experimental import pallas as pl


def kernel(edge_index, batch, emb):
    raise NotImplementedError("write your pallas kernel here")


# speedup vs baseline: 10.2693x; 10.2693x over previous
import functools

import jax
import jax.numpy as jnp
from jax import lax
from jax.experimental import pallas as pl
from jax.experimental.pallas import tpu as pltpu
from jax.experimental.pallas import tpu_sc as plsc

N = 50000
D = 64
H = 32
NLAYERS = 3
ALPHA = 1.0 / (NLAYERS + 1)

NZ = 50048
E = 800000
EP = 802816
EROWS = EP // 128
NT = 16
RPT = NZ // NT
U = 20480

_f32 = jnp.float32
_i32 = jnp.int32


def _zero_vec128(buf):
    for j in range(8):
        buf[pl.ds(j * 16, 16)] = jnp.zeros((16,), _f32)


def _zero_stripe_1d(zbuf, sh, base):
    def body(j, carry):
        pltpu.sync_copy(zbuf, sh.at[pl.ds(base + j * 128, 128)])
        return carry

    lax.fori_loop(0, RPT // 128, body, 0)
    rem = RPT % 128
    if rem:
        pltpu.sync_copy(zbuf.at[pl.ds(0, rem)],
                        sh.at[pl.ds(base + (RPT // 128) * 128, rem)])


def _deg_body(dst2, dga, dgb, idst, ones_v, zbuf, deg_sh):
    c = lax.axis_index("c")
    s = lax.axis_index("s")
    for j in range(8):
        ones_v[pl.ds(j * 16, 16)] = jnp.ones((16,), _f32)
    _zero_vec128(zbuf)
    base = s * RPT
    _zero_stripe_1d(zbuf, deg_sh, base)
    plsc.subcore_barrier()

    nrows = EROWS // 32

    def body(j, carry):
        r = (c * NT + s) * nrows + j
        pltpu.sync_copy(dst2.at[pl.ds(r, 1)], idst)
        pltpu.sync_copy(ones_v, deg_sh.at[idst.at[0]], add=True)
        return carry

    lax.fori_loop(0, nrows, body, 0)
    plsc.subcore_barrier()

    def wout(dg):
        def body(j, carry):
            pltpu.sync_copy(deg_sh.at[pl.ds(base + j * 128, 128)], zbuf)
            pltpu.sync_copy(zbuf, dg.at[pl.ds(base + j * 128, 128)])
            return carry

        lax.fori_loop(0, RPT // 128, body, 0)
        rem = RPT % 128
        if rem:
            off = base + (RPT // 128) * 128
            pltpu.sync_copy(deg_sh.at[pl.ds(off, rem)], zbuf.at[pl.ds(0, rem)])
            pltpu.sync_copy(zbuf.at[pl.ds(0, rem)], dg.at[pl.ds(off, rem)])

    pl.when(c == 0)(lambda: wout(dga))
    pl.when(c == 1)(lambda: wout(dgb))


@functools.lru_cache(maxsize=None)
def _deg_call():
    mesh = plsc.VectorSubcoreMesh(core_axis_name="c", subcore_axis_name="s")
    return pl.kernel(
        _deg_body,
        out_type=[jax.ShapeDtypeStruct((NZ,), _f32)] * 2,
        mesh=mesh,
        compiler_params=pltpu.CompilerParams(use_tc_tiling_on_sc=False),
        scratch_types=[
            pltpu.VMEM((1, 128), _i32),
            pltpu.VMEM((128,), _f32),
            pltpu.VMEM((128,), _f32),
            pltpu.VMEM_SHARED((NZ,), _f32),
        ],
    )


def _spmm_body(src2, dst2, ya, yb, za, zb,
               isrc, idst, rows0, rows1, z_sh, g0, g1):
    c = lax.axis_index("c")
    s = lax.axis_index("s")

    def zrow(i, carry):
        rows0[i, pl.ds(0, 16)] = jnp.zeros((16,), _f32)
        rows0[i, pl.ds(16, 16)] = jnp.zeros((16,), _f32)
        return carry

    lax.fori_loop(0, 128, zrow, 0)
    base = s * RPT

    def zcp(j, carry):
        pltpu.sync_copy(rows0, z_sh.at[pl.ds(base + j * 128, 128)])
        return carry

    lax.fori_loop(0, RPT // 128, zcp, 0)
    rem = RPT % 128
    if rem:
        pltpu.sync_copy(rows0.at[pl.ds(0, rem)],
                        z_sh.at[pl.ds(base + (RPT // 128) * 128, rem)])
    plsc.subcore_barrier()

    nrows = EROWS // NT

    def run(tab, zout):
        def body(j, carry):
            r = s * nrows + 2 * j
            pltpu.sync_copy(src2.at[pl.ds(r, 2)], isrc)
            pltpu.sync_copy(dst2.at[pl.ds(r, 2)], idst)
            d0 = pltpu.async_copy(tab.at[isrc.at[0]], rows0, g0)
            d1 = pltpu.async_copy(tab.at[isrc.at[1]], rows1, g1)
            d0.wait()
            pltpu.sync_copy(rows0, z_sh.at[idst.at[0]], add=True)
            d1.wait()
            pltpu.sync_copy(rows1, z_sh.at[idst.at[1]], add=True)
            return carry

        lax.fori_loop(0, nrows // 2, body, 0)
        plsc.subcore_barrier()

        def wb(j, carry):
            pltpu.sync_copy(z_sh.at[pl.ds(base + j * 128, 128)], rows0)
            pltpu.sync_copy(rows0, zout.at[pl.ds(base + j * 128, 128)])
            return carry

        lax.fori_loop(0, RPT // 128, wb, 0)
        remw = RPT % 128
        if remw:
            offw = base + (RPT // 128) * 128
            pltpu.sync_copy(z_sh.at[pl.ds(offw, remw)],
                            rows0.at[pl.ds(0, remw)])
            pltpu.sync_copy(rows0.at[pl.ds(0, remw)],
                            zout.at[pl.ds(offw, remw)])

    pl.when(c == 0)(lambda: run(ya, za))
    pl.when(c == 1)(lambda: run(yb, zb))


@functools.lru_cache(maxsize=None)
def _spmm_call():
    mesh = plsc.VectorSubcoreMesh(core_axis_name="c", subcore_axis_name="s")
    return pl.kernel(
        _spmm_body,
        out_type=[jax.ShapeDtypeStruct((NZ, H), _f32)] * 2,
        mesh=mesh,
        compiler_params=pltpu.CompilerParams(use_tc_tiling_on_sc=False),
        scratch_types=[
            pltpu.VMEM((2, 128), _i32),
            pltpu.VMEM((2, 128), _i32),
            pltpu.VMEM((128, H), _f32),
            pltpu.VMEM((128, H), _f32),
            pltpu.VMEM_SHARED((NZ, H), _f32),
            pltpu.SemaphoreType.DMA,
            pltpu.SemaphoreType.DMA,
        ],
    )


def _score_body(u2, i2, oa, ob, ua, ub, ia, ib, idxv, rows, sem):
    c = lax.axis_index("c")
    s = lax.axis_index("s")
    w = s * 2 + c
    nrows = U // 128 // 32

    def body(k, carry):
        r = w * nrows + k
        off = r * 128
        pltpu.sync_copy(u2.at[pl.ds(r, 1)], idxv)
        pltpu.async_copy(oa.at[idxv.at[0]], rows, sem).wait()
        pltpu.sync_copy(rows, ua.at[pl.ds(off, 128)])
        pltpu.async_copy(ob.at[idxv.at[0]], rows, sem).wait()
        pltpu.sync_copy(rows, ub.at[pl.ds(off, 128)])
        pltpu.sync_copy(i2.at[pl.ds(r, 1)], idxv)
        pltpu.async_copy(oa.at[idxv.at[0]], rows, sem).wait()
        pltpu.sync_copy(rows, ia.at[pl.ds(off, 128)])
        pltpu.async_copy(ob.at[idxv.at[0]], rows, sem).wait()
        pltpu.sync_copy(rows, ib.at[pl.ds(off, 128)])
        return carry

    lax.fori_loop(0, nrows, body, 0)


@functools.lru_cache(maxsize=None)
def _score_call():
    mesh = plsc.VectorSubcoreMesh(core_axis_name="c", subcore_axis_name="s")
    return pl.kernel(
        _score_body,
        out_type=[jax.ShapeDtypeStruct((U, H), _f32)] * 4,
        mesh=mesh,
        compiler_params=pltpu.CompilerParams(use_tc_tiling_on_sc=False),
        scratch_types=[
            pltpu.VMEM((1, 128), _i32),
            pltpu.VMEM((128, H), _f32),
            pltpu.SemaphoreType.DMA,
        ],
    )


def _dinv_of(deg):
    return jnp.where(deg > 0, lax.rsqrt(deg), jnp.zeros_like(deg))


def _scale_body(power, dga, dgb, za, zb, ya, yb):
    d = _dinv_of(dga[...] + dgb[...])
    if power == 2:
        d = d * d
    ya[...] = za[...] * d
    yb[...] = zb[...] * d


def _make_scale(power):
    G = 16
    R = NZ // G
    return pl.pallas_call(
        functools.partial(_scale_body, power),
        grid=(G,),
        in_specs=[
            pl.BlockSpec((R, 1), lambda i: (i, 0)),
            pl.BlockSpec((R, 1), lambda i: (i, 0)),
            pl.BlockSpec((R, H), lambda i: (i, 0)),
            pl.BlockSpec((R, H), lambda i: (i, 0)),
        ],
        out_specs=[pl.BlockSpec((R, H), lambda i: (i, 0))] * 2,
        out_shape=[jax.ShapeDtypeStruct((NZ, H), _f32)] * 2,
    )


_scale1 = _make_scale(1)
_scale2 = _make_scale(2)


def _final_body(dga, dgb, ea, eb, z1a, z1b, z2a, z2b, z3a, z3b, oa, ob):
    d = _dinv_of(dga[...] + dgb[...])
    oa[...] = ALPHA * (ea[...] + d * (z1a[...] + z2a[...] + z3a[...]))
    ob[...] = ALPHA * (eb[...] + d * (z1b[...] + z2b[...] + z3b[...]))


def _make_final():
    G = 16
    R = NZ // G
    n1 = [pl.BlockSpec((R, 1), lambda i: (i, 0))] * 2
    nh = [pl.BlockSpec((R, H), lambda i: (i, 0))] * 8
    return pl.pallas_call(
        _final_body,
        grid=(G,),
        in_specs=n1 + nh,
        out_specs=[pl.BlockSpec((R, H), lambda i: (i, 0))] * 2,
        out_shape=[jax.ShapeDtypeStruct((NZ, H), _f32)] * 2,
    )


_final_call = _make_final()


def _dot_body(ua, ub, ia, ib, out):
    out[...] = jnp.sum(ua[...] * ia[...] + ub[...] * ib[...],
                       axis=1, keepdims=True)


def _make_dot():
    G = 8
    R = U // G
    return pl.pallas_call(
        _dot_body,
        grid=(G,),
        in_specs=[pl.BlockSpec((R, H), lambda i: (i, 0))] * 4,
        out_specs=pl.BlockSpec((R, 1), lambda i: (i, 0)),
        out_shape=jax.ShapeDtypeStruct((U, 1), _f32),
    )


_dot_call = _make_dot()


def kernel(edge_index, batch, emb):
    src = edge_index[0].astype(_i32)
    dst = edge_index[1].astype(_i32)
    pad = EP - E
    fill = jnp.full((pad,), N, _i32)
    src2 = jnp.concatenate([src, fill]).reshape(EROWS, 128)
    dst2 = jnp.concatenate([dst, fill]).reshape(EROWS, 128)
    embp = jnp.pad(emb, ((0, NZ - N), (0, 0)))
    ea = embp[:, :H]
    eb = embp[:, H:]
    u2 = batch[:, :, 0].reshape(U // 128, 128).astype(_i32)
    i2 = batch[:, :, 1].reshape(U // 128, 128).astype(_i32)

    dga, dgb = _deg_call()(dst2)
    dga1 = dga.reshape(NZ, 1)
    dgb1 = dgb.reshape(NZ, 1)

    spmm = _spmm_call()
    ya, yb = _scale1(dga1, dgb1, ea, eb)
    z1a, z1b = spmm(src2, dst2, ya, yb)
    ya, yb = _scale2(dga1, dgb1, z1a, z1b)
    z2a, z2b = spmm(src2, dst2, ya, yb)
    ya, yb = _scale2(dga1, dgb1, z2a, z2b)
    z3a, z3b = spmm(src2, dst2, ya, yb)

    oa, ob = _final_call(dga1, dgb1, ea, eb, z1a, z1b, z2a, z2b, z3a, z3b)
    ua, ub, ia, ib = _score_call()(u2, i2, oa, ob)
    logits = _dot_call(ua, ub, ia, ib)
    return logits.reshape(batch.shape[0], -1)


# speedup vs baseline: 14.6527x; 1.4269x over previous
import functools

import jax
import jax.numpy as jnp
from jax import lax
from jax.experimental import pallas as pl
from jax.experimental.pallas import tpu as pltpu
from jax.experimental.pallas import tpu_sc as plsc

N = 50000
D = 64
H = 32
NLAYERS = 3
ALPHA = 1.0 / (NLAYERS + 1)

NZ = 50048
E = 800000
EP = 802816
EROWS = EP // 128
NT = 16
RPT = NZ // NT
U = 20480

_f32 = jnp.float32
_i32 = jnp.int32


def _zero_vec128(buf):
    for j in range(8):
        buf[pl.ds(j * 16, 16)] = jnp.zeros((16,), _f32)


def _zero_stripe_1d(zbuf, sh, base):
    def body(j, carry):
        pltpu.sync_copy(zbuf, sh.at[pl.ds(base + j * 128, 128)])
        return carry

    lax.fori_loop(0, RPT // 128, body, 0)
    rem = RPT % 128
    if rem:
        pltpu.sync_copy(zbuf.at[pl.ds(0, rem)],
                        sh.at[pl.ds(base + (RPT // 128) * 128, rem)])


def _deg_body(dst2, dga, dgb, idst, ones_v, zbuf, deg_sh):
    c = lax.axis_index("c")
    s = lax.axis_index("s")
    for j in range(8):
        ones_v[pl.ds(j * 16, 16)] = jnp.ones((16,), _f32)
    _zero_vec128(zbuf)
    base = s * RPT
    _zero_stripe_1d(zbuf, deg_sh, base)
    plsc.subcore_barrier()

    nrows = EROWS // 32

    def body(j, carry):
        r = (c * NT + s) * nrows + j
        pltpu.sync_copy(dst2.at[pl.ds(r, 1)], idst)
        pltpu.sync_copy(ones_v, deg_sh.at[idst.at[0]], add=True)
        return carry

    lax.fori_loop(0, nrows, body, 0)
    plsc.subcore_barrier()

    def wout(dg):
        def body(j, carry):
            pltpu.sync_copy(deg_sh.at[pl.ds(base + j * 128, 128)], zbuf)
            pltpu.sync_copy(zbuf, dg.at[pl.ds(base + j * 128, 128)])
            return carry

        lax.fori_loop(0, RPT // 128, body, 0)
        rem = RPT % 128
        if rem:
            off = base + (RPT // 128) * 128
            pltpu.sync_copy(deg_sh.at[pl.ds(off, rem)], zbuf.at[pl.ds(0, rem)])
            pltpu.sync_copy(zbuf.at[pl.ds(0, rem)], dg.at[pl.ds(off, rem)])

    pl.when(c == 0)(lambda: wout(dga))
    pl.when(c == 1)(lambda: wout(dgb))


@functools.lru_cache(maxsize=None)
def _deg_call():
    mesh = plsc.VectorSubcoreMesh(core_axis_name="c", subcore_axis_name="s")
    return pl.kernel(
        _deg_body,
        out_type=[jax.ShapeDtypeStruct((NZ,), _f32)] * 2,
        mesh=mesh,
        compiler_params=pltpu.CompilerParams(use_tc_tiling_on_sc=False),
        scratch_types=[
            pltpu.VMEM((1, 128), _i32),
            pltpu.VMEM((128,), _f32),
            pltpu.VMEM((128,), _f32),
            pltpu.VMEM_SHARED((NZ,), _f32),
        ],
    )


_C = 112
_Q = 4
_EC = EP // _C
_CPT = _EC // NT
_NG = _CPT // _Q


def _spmm_body(src2, dst2, ya, yb, za, zb,
               isrc, idst, rows, z_sh, gs):
    c = lax.axis_index("c")
    s = lax.axis_index("s")

    def zrow(i, carry):
        rows[0, 0, i, pl.ds(0, 16)] = jnp.zeros((16,), _f32)
        rows[0, 0, i, pl.ds(16, 16)] = jnp.zeros((16,), _f32)
        return carry

    lax.fori_loop(0, _C, zrow, 0)
    base = s * RPT

    def zcp(j, carry):
        pltpu.sync_copy(rows.at[0, 0], z_sh.at[pl.ds(base + j * _C, _C)])
        return carry

    lax.fori_loop(0, RPT // _C, zcp, 0)
    rem = RPT % _C
    if rem:
        pltpu.sync_copy(rows.at[0, 0].at[pl.ds(0, rem)],
                        z_sh.at[pl.ds(base + (RPT // _C) * _C, rem)])
    plsc.subcore_barrier()

    def run(tab):
        r0 = s * _CPT

        def load_idx(g, b):
            pltpu.sync_copy(src2.at[pl.ds(r0 + g * _Q, _Q)], isrc.at[b])
            pltpu.sync_copy(dst2.at[pl.ds(r0 + g * _Q, _Q)], idst.at[b])

        def issue_group(b):
            for k in range(_Q):
                pltpu.async_copy(tab.at[isrc.at[b, k]],
                                 rows.at[b, k], gs.at[b, k])

        def drain_and_scatter(b):
            for k in range(_Q):
                pltpu.make_async_copy(tab.at[pl.ds(0, _C)],
                                      rows.at[b, k], gs.at[b, k]).wait()
                pltpu.sync_copy(rows.at[b, k],
                                z_sh.at[idst.at[b, k]], add=True)

        load_idx(0, 0)
        issue_group(0)

        def body(g, carry):
            b = lax.rem(g, 2)
            nb = 1 - b

            def advance():
                load_idx(g + 1, nb)
                issue_group(nb)

            pl.when(g + 1 < _NG)(advance)
            drain_and_scatter(b)
            return carry

        lax.fori_loop(0, _NG, body, 0)

    def wout(zout):
        plsc.subcore_barrier()

        def wb(j, carry):
            pltpu.sync_copy(z_sh.at[pl.ds(base + j * _C, _C)],
                            rows.at[0, 0])
            pltpu.sync_copy(rows.at[0, 0],
                            zout.at[pl.ds(base + j * _C, _C)])
            return carry

        lax.fori_loop(0, RPT // _C, wb, 0)
        remw = RPT % _C
        if remw:
            offw = base + (RPT // _C) * _C
            pltpu.sync_copy(z_sh.at[pl.ds(offw, remw)],
                            rows.at[0, 0].at[pl.ds(0, remw)])
            pltpu.sync_copy(rows.at[0, 0].at[pl.ds(0, remw)],
                            zout.at[pl.ds(offw, remw)])

    pl.when(c == 0)(lambda: run(ya))
    pl.when(c == 1)(lambda: run(yb))
    pl.when(c == 0)(lambda: wout(za))
    pl.when(c == 1)(lambda: wout(zb))


@functools.lru_cache(maxsize=None)
def _spmm_call():
    mesh = plsc.VectorSubcoreMesh(core_axis_name="c", subcore_axis_name="s")
    return pl.kernel(
        _spmm_body,
        out_type=[jax.ShapeDtypeStruct((NZ, H), _f32)] * 2,
        mesh=mesh,
        compiler_params=pltpu.CompilerParams(use_tc_tiling_on_sc=False),
        scratch_types=[
            pltpu.VMEM((2, _Q, _C), _i32),
            pltpu.VMEM((2, _Q, _C), _i32),
            pltpu.VMEM((2, _Q, _C, H), _f32),
            pltpu.VMEM_SHARED((NZ, H), _f32),
            pltpu.SemaphoreType.DMA((2, _Q)),
        ],
    )


def _score_body(u2, i2, oa, ob, ua, ub, ia, ib, idxv, rows, sem):
    c = lax.axis_index("c")
    s = lax.axis_index("s")
    w = s * 2 + c
    nrows = U // 128 // 32

    def body(k, carry):
        r = w * nrows + k
        off = r * 128
        pltpu.sync_copy(u2.at[pl.ds(r, 1)], idxv)
        pltpu.async_copy(oa.at[idxv.at[0]], rows, sem).wait()
        pltpu.sync_copy(rows, ua.at[pl.ds(off, 128)])
        pltpu.async_copy(ob.at[idxv.at[0]], rows, sem).wait()
        pltpu.sync_copy(rows, ub.at[pl.ds(off, 128)])
        pltpu.sync_copy(i2.at[pl.ds(r, 1)], idxv)
        pltpu.async_copy(oa.at[idxv.at[0]], rows, sem).wait()
        pltpu.sync_copy(rows, ia.at[pl.ds(off, 128)])
        pltpu.async_copy(ob.at[idxv.at[0]], rows, sem).wait()
        pltpu.sync_copy(rows, ib.at[pl.ds(off, 128)])
        return carry

    lax.fori_loop(0, nrows, body, 0)


@functools.lru_cache(maxsize=None)
def _score_call():
    mesh = plsc.VectorSubcoreMesh(core_axis_name="c", subcore_axis_name="s")
    return pl.kernel(
        _score_body,
        out_type=[jax.ShapeDtypeStruct((U, H), _f32)] * 4,
        mesh=mesh,
        compiler_params=pltpu.CompilerParams(use_tc_tiling_on_sc=False),
        scratch_types=[
            pltpu.VMEM((1, 128), _i32),
            pltpu.VMEM((128, H), _f32),
            pltpu.SemaphoreType.DMA,
        ],
    )


def _dinv_of(deg):
    return jnp.where(deg > 0, lax.rsqrt(deg), jnp.zeros_like(deg))


def _scale_body(power, dga, dgb, za, zb, ya, yb):
    d = _dinv_of(dga[...] + dgb[...])
    if power == 2:
        d = d * d
    ya[...] = za[...] * d
    yb[...] = zb[...] * d


def _make_scale(power):
    G = 16
    R = NZ // G
    return pl.pallas_call(
        functools.partial(_scale_body, power),
        grid=(G,),
        in_specs=[
            pl.BlockSpec((R, 1), lambda i: (i, 0)),
            pl.BlockSpec((R, 1), lambda i: (i, 0)),
            pl.BlockSpec((R, H), lambda i: (i, 0)),
            pl.BlockSpec((R, H), lambda i: (i, 0)),
        ],
        out_specs=[pl.BlockSpec((R, H), lambda i: (i, 0))] * 2,
        out_shape=[jax.ShapeDtypeStruct((NZ, H), _f32)] * 2,
    )


_scale1 = _make_scale(1)
_scale2 = _make_scale(2)


def _final_body(dga, dgb, ea, eb, z1a, z1b, z2a, z2b, z3a, z3b, oa, ob):
    d = _dinv_of(dga[...] + dgb[...])
    oa[...] = ALPHA * (ea[...] + d * (z1a[...] + z2a[...] + z3a[...]))
    ob[...] = ALPHA * (eb[...] + d * (z1b[...] + z2b[...] + z3b[...]))


def _make_final():
    G = 16
    R = NZ // G
    n1 = [pl.BlockSpec((R, 1), lambda i: (i, 0))] * 2
    nh = [pl.BlockSpec((R, H), lambda i: (i, 0))] * 8
    return pl.pallas_call(
        _final_body,
        grid=(G,),
        in_specs=n1 + nh,
        out_specs=[pl.BlockSpec((R, H), lambda i: (i, 0))] * 2,
        out_shape=[jax.ShapeDtypeStruct((NZ, H), _f32)] * 2,
    )


_final_call = _make_final()


def _dot_body(ua, ub, ia, ib, out):
    out[...] = jnp.sum(ua[...] * ia[...] + ub[...] * ib[...],
                       axis=1, keepdims=True)


def _make_dot():
    G = 8
    R = U // G
    return pl.pallas_call(
        _dot_body,
        grid=(G,),
        in_specs=[pl.BlockSpec((R, H), lambda i: (i, 0))] * 4,
        out_specs=pl.BlockSpec((R, 1), lambda i: (i, 0)),
        out_shape=jax.ShapeDtypeStruct((U, 1), _f32),
    )


_dot_call = _make_dot()


def kernel(edge_index, batch, emb):
    src = edge_index[0].astype(_i32)
    dst = edge_index[1].astype(_i32)
    pad = EP - E
    fill = jnp.full((pad,), N, _i32)
    src_p = jnp.concatenate([src, fill])
    dst_p = jnp.concatenate([dst, fill])
    dst2 = dst_p.reshape(EROWS, 128)
    src2c = src_p.reshape(_EC, _C)
    dst2c = dst_p.reshape(_EC, _C)
    embp = jnp.pad(emb, ((0, NZ - N), (0, 0)))
    ea = embp[:, :H]
    eb = embp[:, H:]
    u2 = batch[:, :, 0].reshape(U // 128, 128).astype(_i32)
    i2 = batch[:, :, 1].reshape(U // 128, 128).astype(_i32)

    dga, dgb = _deg_call()(dst2)
    dga1 = dga.reshape(NZ, 1)
    dgb1 = dgb.reshape(NZ, 1)

    spmm = _spmm_call()
    ya, yb = _scale1(dga1, dgb1, ea, eb)
    z1a, z1b = spmm(src2c, dst2c, ya, yb)
    ya, yb = _scale2(dga1, dgb1, z1a, z1b)
    z2a, z2b = spmm(src2c, dst2c, ya, yb)
    ya, yb = _scale2(dga1, dgb1, z2a, z2b)
    z3a, z3b = spmm(src2c, dst2c, ya, yb)

    oa, ob = _final_call(dga1, dgb1, ea, eb, z1a, z1b, z2a, z2b, z3a, z3b)
    ua, ub, ia, ib = _score_call()(u2, i2, oa, ob)
    logits = _dot_call(ua, ub, ia, ib)
    return logits.reshape(batch.shape[0], -1)


# speedup vs baseline: 16.0346x; 1.0943x over previous
import functools

import jax
import jax.numpy as jnp
from jax import lax
from jax.experimental import pallas as pl
from jax.experimental.pallas import tpu as pltpu
from jax.experimental.pallas import tpu_sc as plsc

N = 50000
D = 64
H = 32
NLAYERS = 3
ALPHA = 1.0 / (NLAYERS + 1)

NZ = 50048
E = 800000
EP = 802816
EROWS = EP // 128
NT = 16
RPT = NZ // NT
U = 20480
UP = 21504

_C = 112
_Q = 4
_EC = EP // _C
_CPT = _EC // NT
_NG = _CPT // _Q
_SCPT = UP // _C // NT

_f32 = jnp.float32
_i32 = jnp.int32


def _zero_vec128(buf):
    for j in range(8):
        buf[pl.ds(j * 16, 16)] = jnp.zeros((16,), _f32)


def _deg_body(dst2, dga, dgb, idst, ones_v, zbuf, deg_sh):
    c = lax.axis_index("c")
    s = lax.axis_index("s")
    for j in range(8):
        ones_v[pl.ds(j * 16, 16)] = jnp.ones((16,), _f32)
    _zero_vec128(zbuf)
    base = s * RPT

    def zcp(j, carry):
        pltpu.sync_copy(zbuf, deg_sh.at[pl.ds(base + j * 128, 128)])
        return carry

    lax.fori_loop(0, RPT // 128, zcp, 0)
    rem = RPT % 128
    if rem:
        pltpu.sync_copy(zbuf.at[pl.ds(0, rem)],
                        deg_sh.at[pl.ds(base + (RPT // 128) * 128, rem)])
    plsc.subcore_barrier()

    nrows = EROWS // 32

    def body(j, carry):
        r = (c * NT + s) * nrows + j
        pltpu.sync_copy(dst2.at[pl.ds(r, 1)], idst)
        pltpu.sync_copy(ones_v, deg_sh.at[idst.at[0]], add=True)
        return carry

    lax.fori_loop(0, nrows, body, 0)
    plsc.subcore_barrier()

    def wout(dg):
        def body(j, carry):
            pltpu.sync_copy(deg_sh.at[pl.ds(base + j * 128, 128)], zbuf)
            pltpu.sync_copy(zbuf, dg.at[pl.ds(base + j * 128, 128)])
            return carry

        lax.fori_loop(0, RPT // 128, body, 0)
        if rem:
            off = base + (RPT // 128) * 128
            pltpu.sync_copy(deg_sh.at[pl.ds(off, rem)], zbuf.at[pl.ds(0, rem)])
            pltpu.sync_copy(zbuf.at[pl.ds(0, rem)], dg.at[pl.ds(off, rem)])

    pl.when(c == 0)(lambda: wout(dga))
    pl.when(c == 1)(lambda: wout(dgb))


@functools.lru_cache(maxsize=None)
def _deg_call():
    mesh = plsc.VectorSubcoreMesh(core_axis_name="c", subcore_axis_name="s")
    return pl.kernel(
        _deg_body,
        out_type=[jax.ShapeDtypeStruct((NZ,), _f32)] * 2,
        mesh=mesh,
        compiler_params=pltpu.CompilerParams(use_tc_tiling_on_sc=False),
        scratch_types=[
            pltpu.VMEM((1, 128), _i32),
            pltpu.VMEM((128,), _f32),
            pltpu.VMEM((128,), _f32),
            pltpu.VMEM_SHARED((NZ,), _f32),
        ],
    )


def _mega_body(src2, dst2, emb_a, emb_b, y0a, y0b, d2, dv, u2, i2,
               ysa, ysb, zsa, zsb,
               eu_a, zu_a, ei_a, zi_a, eu_b, zu_b, ei_b, zi_b, du, di,
               isrc, idst, rows, dbuf, z_sh, gs):
    c = lax.axis_index("c")
    s = lax.axis_index("s")
    base = s * RPT
    nzc = RPT // _C
    rem = RPT % _C

    def zero_rows00():
        def zrow(i, carry):
            rows[0, 0, i, pl.ds(0, 16)] = jnp.zeros((16,), _f32)
            rows[0, 0, i, pl.ds(16, 16)] = jnp.zeros((16,), _f32)
            return carry

        lax.fori_loop(0, _C, zrow, 0)

    def zero_stripe():
        def zcp(j, carry):
            pltpu.sync_copy(rows.at[0, 0], z_sh.at[pl.ds(base + j * _C, _C)])
            return carry

        lax.fori_loop(0, nzc, zcp, 0)
        if rem:
            pltpu.sync_copy(rows.at[0, 0].at[pl.ds(0, rem)],
                            z_sh.at[pl.ds(base + nzc * _C, rem)])

    def scatter_layer(tab):
        r0 = s * _CPT

        def load_idx(g, b):
            pltpu.sync_copy(src2.at[pl.ds(r0 + g * _Q, _Q)], isrc.at[b])
            pltpu.sync_copy(dst2.at[pl.ds(r0 + g * _Q, _Q)], idst.at[b])

        def issue_group(b):
            for k in range(_Q):
                pltpu.async_copy(tab.at[isrc.at[b, k]],
                                 rows.at[b, k], gs.at[b, k])

        def drain_and_scatter(b):
            for k in range(_Q):
                pltpu.make_async_copy(tab.at[pl.ds(0, _C)],
                                      rows.at[b, k], gs.at[b, k]).wait()
                pltpu.sync_copy(rows.at[b, k],
                                z_sh.at[idst.at[b, k]], add=True)

        load_idx(0, 0)
        issue_group(0)

        def body(g, carry):
            b = lax.rem(g, 2)
            nb = 1 - b

            def advance():
                load_idx(g + 1, nb)
                issue_group(nb)

            pl.when(g + 1 < _NG)(advance)
            drain_and_scatter(b)
            return carry

        lax.fori_loop(0, _NG, body, 0)

    def scale_rows(nr, zslot, yslot):
        def srow(r, carry):
            dvec = plsc.load_gather(dbuf, [jnp.full((16,), r, _i32)])
            yslot[r, pl.ds(0, 16)] = zslot[r, pl.ds(0, 16)] * dvec
            yslot[r, pl.ds(16, 16)] = zslot[r, pl.ds(16, 16)] * dvec
            return carry

        lax.fori_loop(0, nr, srow, 0)

    def add_rows(nr, dst_slot, src_slot):
        def arow(r, carry):
            dst_slot[r, pl.ds(0, 16)] = (dst_slot[r, pl.ds(0, 16)]
                                         + src_slot[r, pl.ds(0, 16)])
            dst_slot[r, pl.ds(16, 16)] = (dst_slot[r, pl.ds(16, 16)]
                                          + src_slot[r, pl.ds(16, 16)])
            return carry

        lax.fori_loop(0, nr, arow, 0)

    def writeout(layer, ys, zs):
        zslot = rows.at[0, 0]
        yslot = rows.at[0, 1]
        sslot = rows.at[0, 2]

        def chunk(off, nr):
            zsl = zslot.at[pl.ds(0, nr)] if nr != _C else zslot
            ysl = yslot.at[pl.ds(0, nr)] if nr != _C else yslot
            ssl = sslot.at[pl.ds(0, nr)] if nr != _C else sslot
            pltpu.sync_copy(z_sh.at[pl.ds(off, nr)], zsl)
            if layer < NLAYERS:
                pltpu.sync_copy(d2.at[pl.ds(off, nr)],
                                dbuf.at[pl.ds(0, nr)])
                scale_rows(nr, zslot, yslot)
                pltpu.sync_copy(ysl, ys.at[pl.ds(off, nr)])
            if layer == 1:
                pltpu.sync_copy(zsl, zs.at[pl.ds(off, nr)])
            else:
                pltpu.sync_copy(zs.at[pl.ds(off, nr)], ssl)
                add_rows(nr, sslot, zslot)
                pltpu.sync_copy(ssl, zs.at[pl.ds(off, nr)])

        def wb(j, carry):
            chunk(base + j * _C, _C)
            return carry

        lax.fori_loop(0, nzc, wb, 0)
        if rem:
            chunk(base + nzc * _C, rem)

    def run(emb_h, y0_h, ys, zs):
        for layer in (1, 2, 3):
            zero_rows00()
            zero_stripe()
            plsc.subcore_barrier()
            scatter_layer(y0_h if layer == 1 else ys)
            plsc.subcore_barrier()
            writeout(layer, ys, zs)
            plsc.subcore_barrier()

    def score(emb_h, zs, out_eu, out_zu, out_ei, out_zi, out_d, d_from_u):
        def chunkk(m, carry):
            r = s * _SCPT + m
            off = r * _C
            uslot = isrc.at[0, 0]
            islot = isrc.at[0, 1]
            pltpu.sync_copy(u2.at[r], uslot)
            pltpu.sync_copy(i2.at[r], islot)
            pltpu.async_copy(emb_h.at[uslot], rows.at[0, 0], gs.at[0, 0])
            pltpu.async_copy(zs.at[uslot], rows.at[0, 1], gs.at[0, 1])
            pltpu.async_copy(emb_h.at[islot], rows.at[0, 2], gs.at[0, 2])
            pltpu.async_copy(zs.at[islot], rows.at[0, 3], gs.at[0, 3])
            dslot = uslot if d_from_u else islot
            pltpu.async_copy(dv.at[dslot], dbuf, gs.at[1, 0])
            pltpu.make_async_copy(emb_h.at[pl.ds(0, _C)], rows.at[0, 0],
                                  gs.at[0, 0]).wait()
            pltpu.sync_copy(rows.at[0, 0], out_eu.at[pl.ds(off, _C)])
            pltpu.make_async_copy(emb_h.at[pl.ds(0, _C)], rows.at[0, 1],
                                  gs.at[0, 1]).wait()
            pltpu.sync_copy(rows.at[0, 1], out_zu.at[pl.ds(off, _C)])
            pltpu.make_async_copy(emb_h.at[pl.ds(0, _C)], rows.at[0, 2],
                                  gs.at[0, 2]).wait()
            pltpu.sync_copy(rows.at[0, 2], out_ei.at[pl.ds(off, _C)])
            pltpu.make_async_copy(emb_h.at[pl.ds(0, _C)], rows.at[0, 3],
                                  gs.at[0, 3]).wait()
            pltpu.sync_copy(rows.at[0, 3], out_zi.at[pl.ds(off, _C)])
            pltpu.make_async_copy(dv.at[pl.ds(0, _C)], dbuf,
                                  gs.at[1, 0]).wait()
            pltpu.sync_copy(dbuf, out_d.at[pl.ds(off, _C)])
            return carry

        lax.fori_loop(0, _SCPT, chunkk, 0)

    def run_a():
        run(emb_a, y0a, ysa, zsa)
        score(emb_a, zsa, eu_a, zu_a, ei_a, zi_a, du, True)

    def run_b():
        run(emb_b, y0b, ysb, zsb)
        score(emb_b, zsb, eu_b, zu_b, ei_b, zi_b, di, False)

    pl.when(c == 0)(run_a)
    pl.when(c == 1)(run_b)


@functools.lru_cache(maxsize=None)
def _mega_call():
    mesh = plsc.VectorSubcoreMesh(core_axis_name="c", subcore_axis_name="s")
    nh = [jax.ShapeDtypeStruct((NZ, H), _f32)] * 4
    gh = [jax.ShapeDtypeStruct((UP, H), _f32)] * 8
    dh = [jax.ShapeDtypeStruct((UP,), _f32)] * 2
    return pl.kernel(
        _mega_body,
        out_type=nh + gh + dh,
        mesh=mesh,
        compiler_params=pltpu.CompilerParams(use_tc_tiling_on_sc=False,
                                             needs_layout_passes=False),
        scratch_types=[
            pltpu.VMEM((2, _Q, _C), _i32),
            pltpu.VMEM((2, _Q, _C), _i32),
            pltpu.VMEM((2, _Q, _C, H), _f32),
            pltpu.VMEM((_C,), _f32),
            pltpu.VMEM_SHARED((NZ, H), _f32),
            pltpu.SemaphoreType.DMA((2, _Q)),
        ],
    )


def _prep_body(dga, dgb, ea, eb, y0a, y0b, dv1, d21):
    deg = dga[...] + dgb[...]
    d = jnp.where(deg > 0, lax.rsqrt(deg), jnp.zeros_like(deg))
    y0a[...] = ea[...] * d
    y0b[...] = eb[...] * d
    dv1[...] = d
    d21[...] = d * d


def _make_prep():
    G = 16
    R = NZ // G
    n1 = pl.BlockSpec((R, 1), lambda i: (i, 0))
    nh = pl.BlockSpec((R, H), lambda i: (i, 0))
    return pl.pallas_call(
        _prep_body,
        grid=(G,),
        in_specs=[n1, n1, nh, nh],
        out_specs=[nh, nh, n1, n1],
        out_shape=[jax.ShapeDtypeStruct((NZ, H), _f32)] * 2
        + [jax.ShapeDtypeStruct((NZ, 1), _f32)] * 2,
    )


_prep_call = _make_prep()


def _dot_body(eu_a, zu_a, ei_a, zi_a, eu_b, zu_b, ei_b, zi_b, du1, di1, out):
    du = du1[...]
    di = di1[...]
    oua = eu_a[...] + du * zu_a[...]
    oub = eu_b[...] + du * zu_b[...]
    oia = ei_a[...] + di * zi_a[...]
    oib = ei_b[...] + di * zi_b[...]
    out[...] = (ALPHA * ALPHA) * jnp.sum(oua * oia + oub * oib,
                                         axis=1, keepdims=True)


def _make_dot():
    G = 8
    R = UP // G
    n1 = pl.BlockSpec((R, 1), lambda i: (i, 0))
    nh = pl.BlockSpec((R, H), lambda i: (i, 0))
    return pl.pallas_call(
        _dot_body,
        grid=(G,),
        in_specs=[nh] * 8 + [n1, n1],
        out_specs=n1,
        out_shape=jax.ShapeDtypeStruct((UP, 1), _f32),
    )


_dot_call = _make_dot()


def kernel(edge_index, batch, emb):
    src = edge_index[0].astype(_i32)
    dst = edge_index[1].astype(_i32)
    pad = EP - E
    fill = jnp.full((pad,), N, _i32)
    src_p = jnp.concatenate([src, fill])
    dst_p = jnp.concatenate([dst, fill])
    dst2 = dst_p.reshape(EROWS, 128)
    src2c = src_p.reshape(_EC, _C)
    dst2c = dst_p.reshape(_EC, _C)
    embp = jnp.pad(emb, ((0, NZ - N), (0, 0)))
    ea = embp[:, :H]
    eb = embp[:, H:]
    ufill = jnp.full((UP - U,), N, _i32)
    u2 = jnp.concatenate([batch[:, :, 0].reshape(-1).astype(_i32),
                          ufill]).reshape(UP // _C, _C)
    i2 = jnp.concatenate([batch[:, :, 1].reshape(-1).astype(_i32),
                          ufill]).reshape(UP // _C, _C)

    dga, dgb = _deg_call()(dst2)
    dga1 = dga.reshape(NZ, 1)
    dgb1 = dgb.reshape(NZ, 1)
    y0a, y0b, dv1, d21 = _prep_call(dga1, dgb1, ea, eb)

    outs = _mega_call()(src2c, dst2c, ea, eb, y0a, y0b,
                        d21.reshape(NZ), dv1.reshape(NZ), u2, i2)
    (_ysa, _ysb, _zsa, _zsb,
     eu_a, zu_a, ei_a, zi_a, eu_b, zu_b, ei_b, zi_b, du, di) = outs

    logits = _dot_call(eu_a, zu_a, ei_a, zi_a, eu_b, zu_b, ei_b, zi_b,
                       du.reshape(UP, 1), di.reshape(UP, 1))
    return logits[:U].reshape(batch.shape[0], -1)


# speedup vs baseline: 16.8383x; 1.0501x over previous
import functools

import jax
import jax.numpy as jnp
from jax import lax
from jax.experimental import pallas as pl
from jax.experimental.pallas import tpu as pltpu
from jax.experimental.pallas import tpu_sc as plsc

N = 50000
D = 64
H = 32
NLAYERS = 3
ALPHA = 1.0 / (NLAYERS + 1)

NZ = 50048
E = 800000
EP = 802816
EROWS = EP // 128
NT = 16
RPT = NZ // NT
U = 20480
UP = 21504

_C = 112
_Q = 4
_EC = EP // _C
_CPT = _EC // NT
_NG = _CPT // _Q
_SCPT = UP // _C // NT

_f32 = jnp.float32
_i32 = jnp.int32


def _zero_vec128(buf):
    for j in range(8):
        buf[pl.ds(j * 16, 16)] = jnp.zeros((16,), _f32)


def _deg_body(dst2, dga, dgb, idst, ones_v, zbuf, deg_sh, dsem):
    c = lax.axis_index("c")
    s = lax.axis_index("s")
    for j in range(8):
        ones_v[pl.ds(j * 16, 16)] = jnp.ones((16,), _f32)
    _zero_vec128(zbuf)
    base = s * RPT

    def zcp(j, carry):
        pltpu.sync_copy(zbuf, deg_sh.at[pl.ds(base + j * 128, 128)])
        return carry

    lax.fori_loop(0, RPT // 128, zcp, 0)
    rem = RPT % 128
    if rem:
        pltpu.sync_copy(zbuf.at[pl.ds(0, rem)],
                        deg_sh.at[pl.ds(base + (RPT // 128) * 128, rem)])
    plsc.subcore_barrier()

    nrows = EROWS // 32
    r0 = (c * NT + s) * nrows
    pltpu.sync_copy(dst2.at[pl.ds(r0, 1)], idst.at[0])

    def body(j, carry):
        b = lax.rem(j, 2)
        nb = 1 - b

        def preload():
            pltpu.async_copy(dst2.at[pl.ds(r0 + j + 1, 1)], idst.at[nb],
                             dsem)

        pl.when(j + 1 < nrows)(preload)
        pltpu.sync_copy(ones_v, deg_sh.at[idst.at[b, 0]], add=True)

        def drain():
            pltpu.make_async_copy(dst2.at[pl.ds(r0, 1)], idst.at[nb],
                                  dsem).wait()

        pl.when(j + 1 < nrows)(drain)
        return carry

    lax.fori_loop(0, nrows, body, 0)
    plsc.subcore_barrier()

    def wout(dg):
        def body(j, carry):
            pltpu.sync_copy(deg_sh.at[pl.ds(base + j * 128, 128)], zbuf)
            pltpu.sync_copy(zbuf, dg.at[pl.ds(base + j * 128, 128)])
            return carry

        lax.fori_loop(0, RPT // 128, body, 0)
        if rem:
            off = base + (RPT // 128) * 128
            pltpu.sync_copy(deg_sh.at[pl.ds(off, rem)], zbuf.at[pl.ds(0, rem)])
            pltpu.sync_copy(zbuf.at[pl.ds(0, rem)], dg.at[pl.ds(off, rem)])

    pl.when(c == 0)(lambda: wout(dga))
    pl.when(c == 1)(lambda: wout(dgb))


@functools.lru_cache(maxsize=None)
def _deg_call():
    mesh = plsc.VectorSubcoreMesh(core_axis_name="c", subcore_axis_name="s")
    return pl.kernel(
        _deg_body,
        out_type=[jax.ShapeDtypeStruct((NZ,), _f32)] * 2,
        mesh=mesh,
        compiler_params=pltpu.CompilerParams(use_tc_tiling_on_sc=False),
        scratch_types=[
            pltpu.VMEM((2, 1, 128), _i32),
            pltpu.VMEM((128,), _f32),
            pltpu.VMEM((128,), _f32),
            pltpu.VMEM_SHARED((NZ,), _f32),
            pltpu.SemaphoreType.DMA,
        ],
    )


def _mega_body(src2, dst2, emb_a, emb_b, y0a, y0b, d2, dv, u2, i2,
               ysa, ysb, zsa, zsb,
               eu_a, zu_a, ei_a, zi_a, eu_b, zu_b, ei_b, zi_b, du, di,
               isrc, idst, rows, dbuf, z_sh, gs):
    c = lax.axis_index("c")
    s = lax.axis_index("s")
    base = s * RPT
    nzc = RPT // _C
    rem = RPT % _C

    def zero_rows00():
        def zrow(i, carry):
            rows[0, 0, i, pl.ds(0, 16)] = jnp.zeros((16,), _f32)
            rows[0, 0, i, pl.ds(16, 16)] = jnp.zeros((16,), _f32)
            return carry

        lax.fori_loop(0, _C, zrow, 0)

    def zero_stripe():
        def zcp(j, carry):
            pltpu.sync_copy(rows.at[0, 0], z_sh.at[pl.ds(base + j * _C, _C)])
            return carry

        lax.fori_loop(0, nzc, zcp, 0)
        if rem:
            pltpu.sync_copy(rows.at[0, 0].at[pl.ds(0, rem)],
                            z_sh.at[pl.ds(base + nzc * _C, rem)])

    def scatter_layer(tab):
        r0 = s * _CPT

        def load_idx(g, b):
            pltpu.sync_copy(src2.at[pl.ds(r0 + g * _Q, _Q)], isrc.at[b])
            pltpu.sync_copy(dst2.at[pl.ds(r0 + g * _Q, _Q)], idst.at[b])

        def issue_group(b):
            for k in range(_Q):
                pltpu.async_copy(tab.at[isrc.at[b, k]],
                                 rows.at[b, k], gs.at[b, k])

        def drain_and_scatter(b):
            for k in range(_Q):
                pltpu.make_async_copy(tab.at[pl.ds(0, _C)],
                                      rows.at[b, k], gs.at[b, k]).wait()
                pltpu.sync_copy(rows.at[b, k],
                                z_sh.at[idst.at[b, k]], add=True)

        load_idx(0, 0)
        issue_group(0)

        def body(g, carry):
            b = lax.rem(g, 2)
            nb = 1 - b

            def advance():
                load_idx(g + 1, nb)
                issue_group(nb)

            pl.when(g + 1 < _NG)(advance)
            drain_and_scatter(b)
            return carry

        lax.fori_loop(0, _NG, body, 0)

    def scale_rows(nr, zslot, yslot, dslot):
        def srow(r, carry):
            dvec = plsc.load_gather(dslot, [jnp.full((16,), r, _i32)])
            yslot[r, pl.ds(0, 16)] = zslot[r, pl.ds(0, 16)] * dvec
            yslot[r, pl.ds(16, 16)] = zslot[r, pl.ds(16, 16)] * dvec
            return carry

        lax.fori_loop(0, nr, srow, 0)

    def add_rows(nr, dst_slot, src_slot):
        def arow(r, carry):
            dst_slot[r, pl.ds(0, 16)] = (dst_slot[r, pl.ds(0, 16)]
                                         + src_slot[r, pl.ds(0, 16)])
            dst_slot[r, pl.ds(16, 16)] = (dst_slot[r, pl.ds(16, 16)]
                                          + src_slot[r, pl.ds(16, 16)])
            return carry

        lax.fori_loop(0, nr, arow, 0)

    def writeout(layer, ys, zs):
        def issue_loads(j, b):
            off = base + j * _C
            pltpu.async_copy(z_sh.at[pl.ds(off, _C)], rows.at[b, 0],
                             gs.at[b, 0])
            if layer < NLAYERS:
                pltpu.async_copy(d2.at[pl.ds(off, _C)], dbuf.at[b],
                                 gs.at[b, 1])
            if layer > 1:
                pltpu.async_copy(zs.at[pl.ds(off, _C)], rows.at[b, 2],
                                 gs.at[b, 2])

        def wait_loads(b):
            pltpu.make_async_copy(z_sh.at[pl.ds(base, _C)], rows.at[b, 0],
                                  gs.at[b, 0]).wait()
            if layer < NLAYERS:
                pltpu.make_async_copy(d2.at[pl.ds(base, _C)], dbuf.at[b],
                                      gs.at[b, 1]).wait()
            if layer > 1:
                pltpu.make_async_copy(zs.at[pl.ds(base, _C)], rows.at[b, 2],
                                      gs.at[b, 2]).wait()

        def wait_stores(b):
            if layer < NLAYERS:
                pltpu.make_async_copy(rows.at[b, 1], ys.at[pl.ds(base, _C)],
                                      gs.at[b, 3]).wait()
            pltpu.make_async_copy(rows.at[b, 2], zs.at[pl.ds(base, _C)],
                                  gs.at[b, 3]).wait()

        def compute_and_store(j, b):
            off = base + j * _C
            if layer < NLAYERS:
                scale_rows(_C, rows.at[b, 0], rows.at[b, 1], dbuf.at[b])
                pltpu.async_copy(rows.at[b, 1], ys.at[pl.ds(off, _C)],
                                 gs.at[b, 3])
            if layer > 1:
                add_rows(_C, rows.at[b, 2], rows.at[b, 0])
                pltpu.async_copy(rows.at[b, 2], zs.at[pl.ds(off, _C)],
                                 gs.at[b, 3])
            else:
                pltpu.async_copy(rows.at[b, 0], zs.at[pl.ds(off, _C)],
                                 gs.at[b, 3])

        issue_loads(0, 0)

        def wb(j, carry):
            b = lax.rem(j, 2)
            nb = 1 - b

            def advance():
                pl.when(j >= 1)(lambda: wait_stores(nb))
                issue_loads(j + 1, nb)

            pl.when(j + 1 < nzc)(advance)
            wait_loads(b)
            compute_and_store(j, b)
            return carry

        lax.fori_loop(0, nzc, wb, 0)
        wait_stores((nzc - 2) % 2)
        wait_stores((nzc - 1) % 2)

        if rem:
            off = base + nzc * _C
            zsl = rows.at[0, 0].at[pl.ds(0, rem)]
            ysl = rows.at[0, 1].at[pl.ds(0, rem)]
            ssl = rows.at[0, 2].at[pl.ds(0, rem)]
            pltpu.sync_copy(z_sh.at[pl.ds(off, rem)], zsl)
            if layer < NLAYERS:
                pltpu.sync_copy(d2.at[pl.ds(off, rem)],
                                dbuf.at[0].at[pl.ds(0, rem)])
                scale_rows(rem, rows.at[0, 0], rows.at[0, 1], dbuf.at[0])
                pltpu.sync_copy(ysl, ys.at[pl.ds(off, rem)])
            if layer == 1:
                pltpu.sync_copy(zsl, zs.at[pl.ds(off, rem)])
            else:
                pltpu.sync_copy(zs.at[pl.ds(off, rem)], ssl)
                add_rows(rem, rows.at[0, 2], rows.at[0, 0])
                pltpu.sync_copy(ssl, zs.at[pl.ds(off, rem)])

    def run(emb_h, y0_h, ys, zs):
        for layer in (1, 2, 3):
            zero_rows00()
            zero_stripe()
            plsc.subcore_barrier()
            scatter_layer(y0_h if layer == 1 else ys)
            plsc.subcore_barrier()
            writeout(layer, ys, zs)
            plsc.subcore_barrier()

    def score(emb_h, zs, out_eu, out_zu, out_ei, out_zi, out_d, d_from_u):
        def chunkk(m, carry):
            r = s * _SCPT + m
            off = r * _C
            uslot = isrc.at[0, 0]
            islot = isrc.at[0, 1]
            pltpu.sync_copy(u2.at[r], uslot)
            pltpu.sync_copy(i2.at[r], islot)
            pltpu.async_copy(emb_h.at[uslot], rows.at[0, 0], gs.at[0, 0])
            pltpu.async_copy(zs.at[uslot], rows.at[0, 1], gs.at[0, 1])
            pltpu.async_copy(emb_h.at[islot], rows.at[0, 2], gs.at[0, 2])
            pltpu.async_copy(zs.at[islot], rows.at[0, 3], gs.at[0, 3])
            dslot = uslot if d_from_u else islot
            pltpu.async_copy(dv.at[dslot], dbuf.at[0], gs.at[1, 0])
            pltpu.make_async_copy(emb_h.at[pl.ds(0, _C)], rows.at[0, 0],
                                  gs.at[0, 0]).wait()
            pltpu.sync_copy(rows.at[0, 0], out_eu.at[pl.ds(off, _C)])
            pltpu.make_async_copy(emb_h.at[pl.ds(0, _C)], rows.at[0, 1],
                                  gs.at[0, 1]).wait()
            pltpu.sync_copy(rows.at[0, 1], out_zu.at[pl.ds(off, _C)])
            pltpu.make_async_copy(emb_h.at[pl.ds(0, _C)], rows.at[0, 2],
                                  gs.at[0, 2]).wait()
            pltpu.sync_copy(rows.at[0, 2], out_ei.at[pl.ds(off, _C)])
            pltpu.make_async_copy(emb_h.at[pl.ds(0, _C)], rows.at[0, 3],
                                  gs.at[0, 3]).wait()
            pltpu.sync_copy(rows.at[0, 3], out_zi.at[pl.ds(off, _C)])
            pltpu.make_async_copy(dv.at[pl.ds(0, _C)], dbuf.at[0],
                                  gs.at[1, 0]).wait()
            pltpu.sync_copy(dbuf.at[0], out_d.at[pl.ds(off, _C)])
            return carry

        lax.fori_loop(0, _SCPT, chunkk, 0)

    def run_a():
        run(emb_a, y0a, ysa, zsa)
        score(emb_a, zsa, eu_a, zu_a, ei_a, zi_a, du, True)

    def run_b():
        run(emb_b, y0b, ysb, zsb)
        score(emb_b, zsb, eu_b, zu_b, ei_b, zi_b, di, False)

    pl.when(c == 0)(run_a)
    pl.when(c == 1)(run_b)


@functools.lru_cache(maxsize=None)
def _mega_call():
    mesh = plsc.VectorSubcoreMesh(core_axis_name="c", subcore_axis_name="s")
    nh = [jax.ShapeDtypeStruct((NZ, H), _f32)] * 4
    gh = [jax.ShapeDtypeStruct((UP, H), _f32)] * 8
    dh = [jax.ShapeDtypeStruct((UP,), _f32)] * 2
    return pl.kernel(
        _mega_body,
        out_type=nh + gh + dh,
        mesh=mesh,
        compiler_params=pltpu.CompilerParams(use_tc_tiling_on_sc=False,
                                             needs_layout_passes=False),
        scratch_types=[
            pltpu.VMEM((2, _Q, _C), _i32),
            pltpu.VMEM((2, _Q, _C), _i32),
            pltpu.VMEM((2, _Q, _C, H), _f32),
            pltpu.VMEM((2, _C), _f32),
            pltpu.VMEM_SHARED((NZ, H), _f32),
            pltpu.SemaphoreType.DMA((2, _Q)),
        ],
    )


def _prep_body(dga, dgb, ea, eb, y0a, y0b, dv1, d21):
    deg = dga[...] + dgb[...]
    d = jnp.where(deg > 0, lax.rsqrt(deg), jnp.zeros_like(deg))
    y0a[...] = ea[...] * d
    y0b[...] = eb[...] * d
    dv1[...] = d
    d21[...] = d * d


def _make_prep():
    G = 16
    R = NZ // G
    n1 = pl.BlockSpec((R, 1), lambda i: (i, 0))
    nh = pl.BlockSpec((R, H), lambda i: (i, 0))
    return pl.pallas_call(
        _prep_body,
        grid=(G,),
        in_specs=[n1, n1, nh, nh],
        out_specs=[nh, nh, n1, n1],
        out_shape=[jax.ShapeDtypeStruct((NZ, H), _f32)] * 2
        + [jax.ShapeDtypeStruct((NZ, 1), _f32)] * 2,
    )


_prep_call = _make_prep()


def _dot_body(eu_a, zu_a, ei_a, zi_a, eu_b, zu_b, ei_b, zi_b, du1, di1, out):
    du = du1[...]
    di = di1[...]
    oua = eu_a[...] + du * zu_a[...]
    oub = eu_b[...] + du * zu_b[...]
    oia = ei_a[...] + di * zi_a[...]
    oib = ei_b[...] + di * zi_b[...]
    out[...] = (ALPHA * ALPHA) * jnp.sum(oua * oia + oub * oib,
                                         axis=1, keepdims=True)


def _make_dot():
    G = 8
    R = UP // G
    n1 = pl.BlockSpec((R, 1), lambda i: (i, 0))
    nh = pl.BlockSpec((R, H), lambda i: (i, 0))
    return pl.pallas_call(
        _dot_body,
        grid=(G,),
        in_specs=[nh] * 8 + [n1, n1],
        out_specs=n1,
        out_shape=jax.ShapeDtypeStruct((UP, 1), _f32),
    )


_dot_call = _make_dot()


def kernel(edge_index, batch, emb):
    src = edge_index[0].astype(_i32)
    dst = edge_index[1].astype(_i32)
    pad = EP - E
    fill = jnp.full((pad,), N, _i32)
    src_p = jnp.concatenate([src, fill])
    dst_p = jnp.concatenate([dst, fill])
    dst2 = dst_p.reshape(EROWS, 128)
    src2c = src_p.reshape(_EC, _C)
    dst2c = dst_p.reshape(_EC, _C)
    embp = jnp.pad(emb, ((0, NZ - N), (0, 0)))
    ea = embp[:, :H]
    eb = embp[:, H:]
    ufill = jnp.full((UP - U,), N, _i32)
    u2 = jnp.concatenate([batch[:, :, 0].reshape(-1).astype(_i32),
                          ufill]).reshape(UP // _C, _C)
    i2 = jnp.concatenate([batch[:, :, 1].reshape(-1).astype(_i32),
                          ufill]).reshape(UP // _C, _C)

    dga, dgb = _deg_call()(dst2)
    dga1 = dga.reshape(NZ, 1)
    dgb1 = dgb.reshape(NZ, 1)
    y0a, y0b, dv1, d21 = _prep_call(dga1, dgb1, ea, eb)

    outs = _mega_call()(src2c, dst2c, ea, eb, y0a, y0b,
                        d21.reshape(NZ), dv1.reshape(NZ), u2, i2)
    (_ysa, _ysb, _zsa, _zsb,
     eu_a, zu_a, ei_a, zi_a, eu_b, zu_b, ei_b, zi_b, du, di) = outs

    logits = _dot_call(eu_a, zu_a, ei_a, zi_a, eu_b, zu_b, ei_b, zi_b,
                       du.reshape(UP, 1), di.reshape(UP, 1))
    return logits[:U].reshape(batch.shape[0], -1)


# speedup vs baseline: 17.5710x; 1.0435x over previous
import functools

import jax
import jax.numpy as jnp
from jax import lax
from jax.experimental import pallas as pl
from jax.experimental.pallas import tpu as pltpu
from jax.experimental.pallas import tpu_sc as plsc

N = 50000
D = 64
H = 32
NLAYERS = 3
ALPHA = 1.0 / (NLAYERS + 1)

NZ = 50048
E = 800000
EP = 802816
EROWS = EP // 128
NT = 16
RPT = NZ // NT
U = 20480
UP = 21504

_C = 112
_Q = 4
_EC = EP // _C
_CPT = _EC // NT
_NG = _CPT // _Q
_SCPT = UP // _C // NT

_f32 = jnp.float32
_i32 = jnp.int32


def _zero_vec128(buf):
    for j in range(8):
        buf[pl.ds(j * 16, 16)] = jnp.zeros((16,), _f32)


def _deg_body(dst2, dga, dgb, idst, ones_v, zbuf, deg_sh, dsem):
    c = lax.axis_index("c")
    s = lax.axis_index("s")
    for j in range(8):
        ones_v[pl.ds(j * 16, 16)] = jnp.ones((16,), _f32)
    _zero_vec128(zbuf)
    base = s * RPT

    def zcp(j, carry):
        pltpu.sync_copy(zbuf, deg_sh.at[pl.ds(base + j * 128, 128)])
        return carry

    lax.fori_loop(0, RPT // 128, zcp, 0)
    rem = RPT % 128
    if rem:
        pltpu.sync_copy(zbuf.at[pl.ds(0, rem)],
                        deg_sh.at[pl.ds(base + (RPT // 128) * 128, rem)])
    plsc.subcore_barrier()

    nrows = EROWS // 32
    r0 = (c * NT + s) * nrows
    pltpu.sync_copy(dst2.at[pl.ds(r0, 1)], idst.at[0])

    def body(j, carry):
        b = lax.rem(j, 2)
        nb = 1 - b

        def preload():
            pltpu.async_copy(dst2.at[pl.ds(r0 + j + 1, 1)], idst.at[nb],
                             dsem)

        pl.when(j + 1 < nrows)(preload)
        pltpu.sync_copy(ones_v, deg_sh.at[idst.at[b, 0]], add=True)

        def drain():
            pltpu.make_async_copy(dst2.at[pl.ds(r0, 1)], idst.at[nb],
                                  dsem).wait()

        pl.when(j + 1 < nrows)(drain)
        return carry

    lax.fori_loop(0, nrows, body, 0)
    plsc.subcore_barrier()

    def wout(dg):
        def body(j, carry):
            pltpu.sync_copy(deg_sh.at[pl.ds(base + j * 128, 128)], zbuf)
            pltpu.sync_copy(zbuf, dg.at[pl.ds(base + j * 128, 128)])
            return carry

        lax.fori_loop(0, RPT // 128, body, 0)
        if rem:
            off = base + (RPT // 128) * 128
            pltpu.sync_copy(deg_sh.at[pl.ds(off, rem)], zbuf.at[pl.ds(0, rem)])
            pltpu.sync_copy(zbuf.at[pl.ds(0, rem)], dg.at[pl.ds(off, rem)])

    pl.when(c == 0)(lambda: wout(dga))
    pl.when(c == 1)(lambda: wout(dgb))


@functools.lru_cache(maxsize=None)
def _deg_call():
    mesh = plsc.VectorSubcoreMesh(core_axis_name="c", subcore_axis_name="s")
    return pl.kernel(
        _deg_body,
        out_type=[jax.ShapeDtypeStruct((NZ,), _f32)] * 2,
        mesh=mesh,
        compiler_params=pltpu.CompilerParams(use_tc_tiling_on_sc=False),
        scratch_types=[
            pltpu.VMEM((2, 1, 128), _i32),
            pltpu.VMEM((128,), _f32),
            pltpu.VMEM((128,), _f32),
            pltpu.VMEM_SHARED((NZ,), _f32),
            pltpu.SemaphoreType.DMA,
        ],
    )


def _mega_body(src2, dst2, emb_a, emb_b, y0a, y0b, d2, dv, u2, i2,
               ysa, ysb, zsa, zsb,
               eu_a, zu_a, ei_a, zi_a, eu_b, zu_b, ei_b, zi_b, du, di,
               isrc, idst, rows, dbuf, z_sh, gs, ss):
    c = lax.axis_index("c")
    s = lax.axis_index("s")
    base = s * RPT
    nzc = RPT // _C
    rem = RPT % _C

    def zero_rows00():
        def zrow(i, carry):
            rows[0, 0, i, pl.ds(0, 16)] = jnp.zeros((16,), _f32)
            rows[0, 0, i, pl.ds(16, 16)] = jnp.zeros((16,), _f32)
            return carry

        lax.fori_loop(0, _C, zrow, 0)

    def zero_stripe():
        def zcp(j, carry):
            pltpu.sync_copy(rows.at[0, 0], z_sh.at[pl.ds(base + j * _C, _C)])
            return carry

        lax.fori_loop(0, nzc, zcp, 0)
        if rem:
            pltpu.sync_copy(rows.at[0, 0].at[pl.ds(0, rem)],
                            z_sh.at[pl.ds(base + nzc * _C, rem)])

    def scatter_layer(tab):
        r0 = s * _CPT

        def load_idx(g, b):
            pltpu.sync_copy(src2.at[pl.ds(r0 + g * _Q, _Q)], isrc.at[b])
            pltpu.sync_copy(dst2.at[pl.ds(r0 + g * _Q, _Q)], idst.at[b])

        def issue_group(b):
            for k in range(_Q):
                pltpu.async_copy(tab.at[isrc.at[b, k]],
                                 rows.at[b, k], gs.at[b, k])

        def wait_scatters(b):
            for k in range(_Q):
                pltpu.make_async_copy(rows.at[b, k],
                                      z_sh.at[pl.ds(0, _C)],
                                      ss.at[b, k]).wait()

        def drain_and_scatter(b):
            for k in range(_Q):
                pltpu.make_async_copy(tab.at[pl.ds(0, _C)],
                                      rows.at[b, k], gs.at[b, k]).wait()
                pltpu.async_copy(rows.at[b, k],
                                 z_sh.at[idst.at[b, k]], ss.at[b, k],
                                 add=True)

        load_idx(0, 0)
        issue_group(0)

        def body(g, carry):
            b = lax.rem(g, 2)
            nb = 1 - b

            def advance():
                pl.when(g >= 1)(lambda: wait_scatters(nb))
                load_idx(g + 1, nb)
                issue_group(nb)

            pl.when(g + 1 < _NG)(advance)
            drain_and_scatter(b)
            return carry

        lax.fori_loop(0, _NG, body, 0)
        wait_scatters((_NG - 2) % 2)
        wait_scatters((_NG - 1) % 2)

    def scale_rows(nr, zslot, yslot, dslot):
        def srow(r, carry):
            dvec = plsc.load_gather(dslot, [jnp.full((16,), r, _i32)])
            yslot[r, pl.ds(0, 16)] = zslot[r, pl.ds(0, 16)] * dvec
            yslot[r, pl.ds(16, 16)] = zslot[r, pl.ds(16, 16)] * dvec
            return carry

        lax.fori_loop(0, nr, srow, 0)

    def add_rows(nr, dst_slot, src_slot):
        def arow(r, carry):
            dst_slot[r, pl.ds(0, 16)] = (dst_slot[r, pl.ds(0, 16)]
                                         + src_slot[r, pl.ds(0, 16)])
            dst_slot[r, pl.ds(16, 16)] = (dst_slot[r, pl.ds(16, 16)]
                                          + src_slot[r, pl.ds(16, 16)])
            return carry

        lax.fori_loop(0, nr, arow, 0)

    def writeout(layer, ys, zs):
        def issue_loads(j, b):
            off = base + j * _C
            pltpu.async_copy(z_sh.at[pl.ds(off, _C)], rows.at[b, 0],
                             gs.at[b, 0])
            if layer < NLAYERS:
                pltpu.async_copy(d2.at[pl.ds(off, _C)], dbuf.at[b],
                                 gs.at[b, 1])
            if layer > 1:
                pltpu.async_copy(zs.at[pl.ds(off, _C)], rows.at[b, 2],
                                 gs.at[b, 2])

        def wait_loads(b):
            pltpu.make_async_copy(z_sh.at[pl.ds(base, _C)], rows.at[b, 0],
                                  gs.at[b, 0]).wait()
            if layer < NLAYERS:
                pltpu.make_async_copy(d2.at[pl.ds(base, _C)], dbuf.at[b],
                                      gs.at[b, 1]).wait()
            if layer > 1:
                pltpu.make_async_copy(zs.at[pl.ds(base, _C)], rows.at[b, 2],
                                      gs.at[b, 2]).wait()

        def wait_stores(b):
            if layer < NLAYERS:
                pltpu.make_async_copy(rows.at[b, 1], ys.at[pl.ds(base, _C)],
                                      gs.at[b, 3]).wait()
            pltpu.make_async_copy(rows.at[b, 2], zs.at[pl.ds(base, _C)],
                                  gs.at[b, 3]).wait()

        def compute_and_store(j, b):
            off = base + j * _C
            if layer < NLAYERS:
                scale_rows(_C, rows.at[b, 0], rows.at[b, 1], dbuf.at[b])
                pltpu.async_copy(rows.at[b, 1], ys.at[pl.ds(off, _C)],
                                 gs.at[b, 3])
            if layer > 1:
                add_rows(_C, rows.at[b, 2], rows.at[b, 0])
                pltpu.async_copy(rows.at[b, 2], zs.at[pl.ds(off, _C)],
                                 gs.at[b, 3])
            else:
                pltpu.async_copy(rows.at[b, 0], zs.at[pl.ds(off, _C)],
                                 gs.at[b, 3])

        issue_loads(0, 0)

        def wb(j, carry):
            b = lax.rem(j, 2)
            nb = 1 - b

            def advance():
                pl.when(j >= 1)(lambda: wait_stores(nb))
                issue_loads(j + 1, nb)

            pl.when(j + 1 < nzc)(advance)
            wait_loads(b)
            compute_and_store(j, b)
            return carry

        lax.fori_loop(0, nzc, wb, 0)
        wait_stores((nzc - 2) % 2)
        wait_stores((nzc - 1) % 2)

        if rem:
            off = base + nzc * _C
            zsl = rows.at[0, 0].at[pl.ds(0, rem)]
            ysl = rows.at[0, 1].at[pl.ds(0, rem)]
            ssl = rows.at[0, 2].at[pl.ds(0, rem)]
            pltpu.sync_copy(z_sh.at[pl.ds(off, rem)], zsl)
            if layer < NLAYERS:
                pltpu.sync_copy(d2.at[pl.ds(off, rem)],
                                dbuf.at[0].at[pl.ds(0, rem)])
                scale_rows(rem, rows.at[0, 0], rows.at[0, 1], dbuf.at[0])
                pltpu.sync_copy(ysl, ys.at[pl.ds(off, rem)])
            if layer == 1:
                pltpu.sync_copy(zsl, zs.at[pl.ds(off, rem)])
            else:
                pltpu.sync_copy(zs.at[pl.ds(off, rem)], ssl)
                add_rows(rem, rows.at[0, 2], rows.at[0, 0])
                pltpu.sync_copy(ssl, zs.at[pl.ds(off, rem)])

    def run(emb_h, y0_h, ys, zs):
        for layer in (1, 2, 3):
            zero_rows00()
            zero_stripe()
            plsc.subcore_barrier()
            scatter_layer(y0_h if layer == 1 else ys)
            plsc.subcore_barrier()
            writeout(layer, ys, zs)
            plsc.subcore_barrier()

    def score(emb_h, zs, out_eu, out_zu, out_ei, out_zi, out_d, d_from_u):
        def chunkk(m, carry):
            r = s * _SCPT + m
            off = r * _C
            uslot = isrc.at[0, 0]
            islot = isrc.at[0, 1]
            pltpu.sync_copy(u2.at[r], uslot)
            pltpu.sync_copy(i2.at[r], islot)
            pltpu.async_copy(emb_h.at[uslot], rows.at[0, 0], gs.at[0, 0])
            pltpu.async_copy(zs.at[uslot], rows.at[0, 1], gs.at[0, 1])
            pltpu.async_copy(emb_h.at[islot], rows.at[0, 2], gs.at[0, 2])
            pltpu.async_copy(zs.at[islot], rows.at[0, 3], gs.at[0, 3])
            dslot = uslot if d_from_u else islot
            pltpu.async_copy(dv.at[dslot], dbuf.at[0], gs.at[1, 0])
            pltpu.make_async_copy(emb_h.at[pl.ds(0, _C)], rows.at[0, 0],
                                  gs.at[0, 0]).wait()
            pltpu.sync_copy(rows.at[0, 0], out_eu.at[pl.ds(off, _C)])
            pltpu.make_async_copy(emb_h.at[pl.ds(0, _C)], rows.at[0, 1],
                                  gs.at[0, 1]).wait()
            pltpu.sync_copy(rows.at[0, 1], out_zu.at[pl.ds(off, _C)])
            pltpu.make_async_copy(emb_h.at[pl.ds(0, _C)], rows.at[0, 2],
                                  gs.at[0, 2]).wait()
            pltpu.sync_copy(rows.at[0, 2], out_ei.at[pl.ds(off, _C)])
            pltpu.make_async_copy(emb_h.at[pl.ds(0, _C)], rows.at[0, 3],
                                  gs.at[0, 3]).wait()
            pltpu.sync_copy(rows.at[0, 3], out_zi.at[pl.ds(off, _C)])
            pltpu.make_async_copy(dv.at[pl.ds(0, _C)], dbuf.at[0],
                                  gs.at[1, 0]).wait()
            pltpu.sync_copy(dbuf.at[0], out_d.at[pl.ds(off, _C)])
            return carry

        lax.fori_loop(0, _SCPT, chunkk, 0)

    def run_a():
        run(emb_a, y0a, ysa, zsa)
        score(emb_a, zsa, eu_a, zu_a, ei_a, zi_a, du, True)

    def run_b():
        run(emb_b, y0b, ysb, zsb)
        score(emb_b, zsb, eu_b, zu_b, ei_b, zi_b, di, False)

    pl.when(c == 0)(run_a)
    pl.when(c == 1)(run_b)


@functools.lru_cache(maxsize=None)
def _mega_call():
    mesh = plsc.VectorSubcoreMesh(core_axis_name="c", subcore_axis_name="s")
    nh = [jax.ShapeDtypeStruct((NZ, H), _f32)] * 4
    gh = [jax.ShapeDtypeStruct((UP, H), _f32)] * 8
    dh = [jax.ShapeDtypeStruct((UP,), _f32)] * 2
    return pl.kernel(
        _mega_body,
        out_type=nh + gh + dh,
        mesh=mesh,
        compiler_params=pltpu.CompilerParams(use_tc_tiling_on_sc=False,
                                             needs_layout_passes=False),
        scratch_types=[
            pltpu.VMEM((2, _Q, _C), _i32),
            pltpu.VMEM((2, _Q, _C), _i32),
            pltpu.VMEM((2, _Q, _C, H), _f32),
            pltpu.VMEM((2, _C), _f32),
            pltpu.VMEM_SHARED((NZ, H), _f32),
            pltpu.SemaphoreType.DMA((2, _Q)),
            pltpu.SemaphoreType.DMA((2, _Q)),
        ],
    )


def _prep_body(dga, dgb, ea, eb, y0a, y0b, dv1, d21):
    deg = dga[...] + dgb[...]
    d = jnp.where(deg > 0, lax.rsqrt(deg), jnp.zeros_like(deg))
    y0a[...] = ea[...] * d
    y0b[...] = eb[...] * d
    dv1[...] = d
    d21[...] = d * d


def _make_prep():
    G = 16
    R = NZ // G
    n1 = pl.BlockSpec((R, 1), lambda i: (i, 0))
    nh = pl.BlockSpec((R, H), lambda i: (i, 0))
    return pl.pallas_call(
        _prep_body,
        grid=(G,),
        in_specs=[n1, n1, nh, nh],
        out_specs=[nh, nh, n1, n1],
        out_shape=[jax.ShapeDtypeStruct((NZ, H), _f32)] * 2
        + [jax.ShapeDtypeStruct((NZ, 1), _f32)] * 2,
    )


_prep_call = _make_prep()


def _dot_body(eu_a, zu_a, ei_a, zi_a, eu_b, zu_b, ei_b, zi_b, du1, di1, out):
    du = du1[...]
    di = di1[...]
    oua = eu_a[...] + du * zu_a[...]
    oub = eu_b[...] + du * zu_b[...]
    oia = ei_a[...] + di * zi_a[...]
    oib = ei_b[...] + di * zi_b[...]
    out[...] = (ALPHA * ALPHA) * jnp.sum(oua * oia + oub * oib,
                                         axis=1, keepdims=True)


def _make_dot():
    G = 8
    R = UP // G
    n1 = pl.BlockSpec((R, 1), lambda i: (i, 0))
    nh = pl.BlockSpec((R, H), lambda i: (i, 0))
    return pl.pallas_call(
        _dot_body,
        grid=(G,),
        in_specs=[nh] * 8 + [n1, n1],
        out_specs=n1,
        out_shape=jax.ShapeDtypeStruct((UP, 1), _f32),
    )


_dot_call = _make_dot()


def kernel(edge_index, batch, emb):
    src = edge_index[0].astype(_i32)
    dst = edge_index[1].astype(_i32)
    pad = EP - E
    fill = jnp.full((pad,), N, _i32)
    src_p = jnp.concatenate([src, fill])
    dst_p = jnp.concatenate([dst, fill])
    dst2 = dst_p.reshape(EROWS, 128)
    src2c = src_p.reshape(_EC, _C)
    dst2c = dst_p.reshape(_EC, _C)
    embp = jnp.pad(emb, ((0, NZ - N), (0, 0)))
    ea = embp[:, :H]
    eb = embp[:, H:]
    ufill = jnp.full((UP - U,), N, _i32)
    u2 = jnp.concatenate([batch[:, :, 0].reshape(-1).astype(_i32),
                          ufill]).reshape(UP // _C, _C)
    i2 = jnp.concatenate([batch[:, :, 1].reshape(-1).astype(_i32),
                          ufill]).reshape(UP // _C, _C)

    dga, dgb = _deg_call()(dst2)
    dga1 = dga.reshape(NZ, 1)
    dgb1 = dgb.reshape(NZ, 1)
    y0a, y0b, dv1, d21 = _prep_call(dga1, dgb1, ea, eb)

    outs = _mega_call()(src2c, dst2c, ea, eb, y0a, y0b,
                        d21.reshape(NZ), dv1.reshape(NZ), u2, i2)
    (_ysa, _ysb, _zsa, _zsb,
     eu_a, zu_a, ei_a, zi_a, eu_b, zu_b, ei_b, zi_b, du, di) = outs

    logits = _dot_call(eu_a, zu_a, ei_a, zi_a, eu_b, zu_b, ei_b, zi_b,
                       du.reshape(UP, 1), di.reshape(UP, 1))
    return logits[:U].reshape(batch.shape[0], -1)


# speedup vs baseline: 18.2381x; 1.0380x over previous
import functools

import jax
import jax.numpy as jnp
from jax import lax
from jax.experimental import pallas as pl
from jax.experimental.pallas import tpu as pltpu
from jax.experimental.pallas import tpu_sc as plsc

N = 50000
D = 64
H = 32
NLAYERS = 3
ALPHA = 1.0 / (NLAYERS + 1)

NZ = 50048
E = 800000
EP = 802816
EROWS = EP // 128
NT = 16
RPT = NZ // NT
U = 20480
UP = 21504

_C = 112
_Q = 4
_EC = EP // _C
_CPT = _EC // NT
_NG = _CPT // _Q
_SCPT = UP // _C // NT

_f32 = jnp.float32
_i32 = jnp.int32


def _zero_vec128(buf):
    for j in range(8):
        buf[pl.ds(j * 16, 16)] = jnp.zeros((16,), _f32)


def _deg_body(dst2, dga, dgb, idst, ones_v, zbuf, deg_sh, dsem, ssd):
    c = lax.axis_index("c")
    s = lax.axis_index("s")
    for j in range(8):
        ones_v[pl.ds(j * 16, 16)] = jnp.ones((16,), _f32)
    _zero_vec128(zbuf)
    base = s * RPT

    def zcp(j, carry):
        pltpu.sync_copy(zbuf, deg_sh.at[pl.ds(base + j * 128, 128)])
        return carry

    lax.fori_loop(0, RPT // 128, zcp, 0)
    rem = RPT % 128
    if rem:
        pltpu.sync_copy(zbuf.at[pl.ds(0, rem)],
                        deg_sh.at[pl.ds(base + (RPT // 128) * 128, rem)])
    plsc.subcore_barrier()

    nrows = EROWS // 32
    r0 = (c * NT + s) * nrows
    for t in range(3):
        pltpu.async_copy(dst2.at[pl.ds(r0 + t, 1)], idst.at[t], dsem.at[t])

    def body(j, carry):
        p = lax.rem(j, 6)
        q = lax.rem(j + 3, 6)

        def drain3():
            pltpu.make_async_copy(ones_v, deg_sh.at[pl.ds(0, 128)],
                                  ssd.at[q]).wait()

        pl.when(j >= 3)(drain3)

        def preload():
            pltpu.async_copy(dst2.at[pl.ds(r0 + j + 3, 1)], idst.at[q],
                             dsem.at[q])

        pl.when(j + 3 < nrows)(preload)
        pltpu.make_async_copy(dst2.at[pl.ds(r0, 1)], idst.at[p],
                              dsem.at[p]).wait()
        pltpu.async_copy(ones_v, deg_sh.at[idst.at[p, 0]], ssd.at[p],
                         add=True)
        return carry

    lax.fori_loop(0, nrows, body, 0)
    for t in (nrows - 3, nrows - 2, nrows - 1):
        pltpu.make_async_copy(ones_v, deg_sh.at[pl.ds(0, 128)],
                              ssd.at[t % 6]).wait()
    plsc.subcore_barrier()

    def wout(dg):
        def body(j, carry):
            pltpu.sync_copy(deg_sh.at[pl.ds(base + j * 128, 128)], zbuf)
            pltpu.sync_copy(zbuf, dg.at[pl.ds(base + j * 128, 128)])
            return carry

        lax.fori_loop(0, RPT // 128, body, 0)
        if rem:
            off = base + (RPT // 128) * 128
            pltpu.sync_copy(deg_sh.at[pl.ds(off, rem)], zbuf.at[pl.ds(0, rem)])
            pltpu.sync_copy(zbuf.at[pl.ds(0, rem)], dg.at[pl.ds(off, rem)])

    pl.when(c == 0)(lambda: wout(dga))
    pl.when(c == 1)(lambda: wout(dgb))


@functools.lru_cache(maxsize=None)
def _deg_call():
    mesh = plsc.VectorSubcoreMesh(core_axis_name="c", subcore_axis_name="s")
    return pl.kernel(
        _deg_body,
        out_type=[jax.ShapeDtypeStruct((NZ,), _f32)] * 2,
        mesh=mesh,
        compiler_params=pltpu.CompilerParams(use_tc_tiling_on_sc=False),
        scratch_types=[
            pltpu.VMEM((6, 1, 128), _i32),
            pltpu.VMEM((128,), _f32),
            pltpu.VMEM((128,), _f32),
            pltpu.VMEM_SHARED((NZ,), _f32),
            pltpu.SemaphoreType.DMA((6,)),
            pltpu.SemaphoreType.DMA((6,)),
        ],
    )


def _mega_body(src2, dst2, emb_a, emb_b, y0a, y0b, d2, dv, u2, i2,
               ysa, ysb, zsa, zsb,
               eu_a, zu_a, ei_a, zi_a, eu_b, zu_b, ei_b, zi_b, du, di,
               isrc, idst, rows, dbuf, z_sh, gs, ss):
    c = lax.axis_index("c")
    s = lax.axis_index("s")
    base = s * RPT
    nzc = RPT // _C
    rem = RPT % _C

    def zero_rows00():
        def zrow(i, carry):
            rows[0, 0, i, pl.ds(0, 16)] = jnp.zeros((16,), _f32)
            rows[0, 0, i, pl.ds(16, 16)] = jnp.zeros((16,), _f32)
            return carry

        lax.fori_loop(0, _C, zrow, 0)

    def zero_stripe():
        def zcp(j, carry):
            pltpu.sync_copy(rows.at[0, 0], z_sh.at[pl.ds(base + j * _C, _C)])
            return carry

        lax.fori_loop(0, nzc, zcp, 0)
        if rem:
            pltpu.sync_copy(rows.at[0, 0].at[pl.ds(0, rem)],
                            z_sh.at[pl.ds(base + nzc * _C, rem)])

    def scatter_layer(tab):
        r0 = s * _CPT

        def load_idx(g, b):
            pltpu.sync_copy(src2.at[pl.ds(r0 + g * _Q, _Q)], isrc.at[b])
            pltpu.sync_copy(dst2.at[pl.ds(r0 + g * _Q, _Q)], idst.at[b])

        def issue_group(b):
            for k in range(_Q):
                pltpu.async_copy(tab.at[isrc.at[b, k]],
                                 rows.at[b, k], gs.at[b, k])

        def wait_scatters(b):
            for k in range(_Q):
                pltpu.make_async_copy(rows.at[b, k],
                                      z_sh.at[pl.ds(0, _C)],
                                      ss.at[b, k]).wait()

        def drain_and_scatter(b):
            for k in range(_Q):
                pltpu.make_async_copy(tab.at[pl.ds(0, _C)],
                                      rows.at[b, k], gs.at[b, k]).wait()
                pltpu.async_copy(rows.at[b, k],
                                 z_sh.at[idst.at[b, k]], ss.at[b, k],
                                 add=True)

        load_idx(0, 0)
        issue_group(0)

        def body(g, carry):
            b = lax.rem(g, 2)
            nb = 1 - b

            def advance():
                pl.when(g >= 1)(lambda: wait_scatters(nb))
                load_idx(g + 1, nb)
                issue_group(nb)

            pl.when(g + 1 < _NG)(advance)
            drain_and_scatter(b)
            return carry

        lax.fori_loop(0, _NG, body, 0)
        wait_scatters((_NG - 2) % 2)
        wait_scatters((_NG - 1) % 2)

    def scale_rows(nr, zslot, yslot, dslot):
        def srow(r, carry):
            dvec = plsc.load_gather(dslot, [jnp.full((16,), r, _i32)])
            yslot[r, pl.ds(0, 16)] = zslot[r, pl.ds(0, 16)] * dvec
            yslot[r, pl.ds(16, 16)] = zslot[r, pl.ds(16, 16)] * dvec
            return carry

        lax.fori_loop(0, nr, srow, 0)

    def add_rows(nr, dst_slot, src_slot):
        def arow(r, carry):
            dst_slot[r, pl.ds(0, 16)] = (dst_slot[r, pl.ds(0, 16)]
                                         + src_slot[r, pl.ds(0, 16)])
            dst_slot[r, pl.ds(16, 16)] = (dst_slot[r, pl.ds(16, 16)]
                                          + src_slot[r, pl.ds(16, 16)])
            return carry

        lax.fori_loop(0, nr, arow, 0)

    def writeout(layer, ys, zs):
        def issue_loads(j, b):
            off = base + j * _C
            pltpu.async_copy(z_sh.at[pl.ds(off, _C)], rows.at[b, 0],
                             gs.at[b, 0])
            if layer < NLAYERS:
                pltpu.async_copy(d2.at[pl.ds(off, _C)], dbuf.at[b],
                                 gs.at[b, 1])
            if layer > 1:
                pltpu.async_copy(zs.at[pl.ds(off, _C)], rows.at[b, 2],
                                 gs.at[b, 2])

        def wait_loads(b):
            pltpu.make_async_copy(z_sh.at[pl.ds(base, _C)], rows.at[b, 0],
                                  gs.at[b, 0]).wait()
            if layer < NLAYERS:
                pltpu.make_async_copy(d2.at[pl.ds(base, _C)], dbuf.at[b],
                                      gs.at[b, 1]).wait()
            if layer > 1:
                pltpu.make_async_copy(zs.at[pl.ds(base, _C)], rows.at[b, 2],
                                      gs.at[b, 2]).wait()

        def wait_stores(b):
            if layer < NLAYERS:
                pltpu.make_async_copy(rows.at[b, 1], ys.at[pl.ds(base, _C)],
                                      gs.at[b, 3]).wait()
            pltpu.make_async_copy(rows.at[b, 2], zs.at[pl.ds(base, _C)],
                                  gs.at[b, 3]).wait()

        def compute_and_store(j, b):
            off = base + j * _C
            if layer < NLAYERS:
                scale_rows(_C, rows.at[b, 0], rows.at[b, 1], dbuf.at[b])
                pltpu.async_copy(rows.at[b, 1], ys.at[pl.ds(off, _C)],
                                 gs.at[b, 3])
            if layer > 1:
                add_rows(_C, rows.at[b, 2], rows.at[b, 0])
                pltpu.async_copy(rows.at[b, 2], zs.at[pl.ds(off, _C)],
                                 gs.at[b, 3])
            else:
                pltpu.async_copy(rows.at[b, 0], zs.at[pl.ds(off, _C)],
                                 gs.at[b, 3])

        issue_loads(0, 0)

        def wb(j, carry):
            b = lax.rem(j, 2)
            nb = 1 - b

            def advance():
                pl.when(j >= 1)(lambda: wait_stores(nb))
                issue_loads(j + 1, nb)

            pl.when(j + 1 < nzc)(advance)
            wait_loads(b)
            compute_and_store(j, b)
            return carry

        lax.fori_loop(0, nzc, wb, 0)
        wait_stores((nzc - 2) % 2)
        wait_stores((nzc - 1) % 2)

        if rem:
            off = base + nzc * _C
            zsl = rows.at[0, 0].at[pl.ds(0, rem)]
            ysl = rows.at[0, 1].at[pl.ds(0, rem)]
            ssl = rows.at[0, 2].at[pl.ds(0, rem)]
            pltpu.sync_copy(z_sh.at[pl.ds(off, rem)], zsl)
            if layer < NLAYERS:
                pltpu.sync_copy(d2.at[pl.ds(off, rem)],
                                dbuf.at[0].at[pl.ds(0, rem)])
                scale_rows(rem, rows.at[0, 0], rows.at[0, 1], dbuf.at[0])
                pltpu.sync_copy(ysl, ys.at[pl.ds(off, rem)])
            if layer == 1:
                pltpu.sync_copy(zsl, zs.at[pl.ds(off, rem)])
            else:
                pltpu.sync_copy(zs.at[pl.ds(off, rem)], ssl)
                add_rows(rem, rows.at[0, 2], rows.at[0, 0])
                pltpu.sync_copy(ssl, zs.at[pl.ds(off, rem)])

    def run(emb_h, y0_h, ys, zs):
        for layer in (1, 2, 3):
            zero_rows00()
            zero_stripe()
            plsc.subcore_barrier()
            scatter_layer(y0_h if layer == 1 else ys)
            plsc.subcore_barrier()
            writeout(layer, ys, zs)
            plsc.subcore_barrier()

    def score(emb_h, zs, out_eu, out_zu, out_ei, out_zi, out_d, d_from_u):
        def chunkk(m, carry):
            r = s * _SCPT + m
            off = r * _C
            uslot = isrc.at[0, 0]
            islot = isrc.at[0, 1]
            pltpu.sync_copy(u2.at[r], uslot)
            pltpu.sync_copy(i2.at[r], islot)
            pltpu.async_copy(emb_h.at[uslot], rows.at[0, 0], gs.at[0, 0])
            pltpu.async_copy(zs.at[uslot], rows.at[0, 1], gs.at[0, 1])
            pltpu.async_copy(emb_h.at[islot], rows.at[0, 2], gs.at[0, 2])
            pltpu.async_copy(zs.at[islot], rows.at[0, 3], gs.at[0, 3])
            dslot = uslot if d_from_u else islot
            pltpu.async_copy(dv.at[dslot], dbuf.at[0], gs.at[1, 0])
            pltpu.make_async_copy(emb_h.at[pl.ds(0, _C)], rows.at[0, 0],
                                  gs.at[0, 0]).wait()
            pltpu.sync_copy(rows.at[0, 0], out_eu.at[pl.ds(off, _C)])
            pltpu.make_async_copy(emb_h.at[pl.ds(0, _C)], rows.at[0, 1],
                                  gs.at[0, 1]).wait()
            pltpu.sync_copy(rows.at[0, 1], out_zu.at[pl.ds(off, _C)])
            pltpu.make_async_copy(emb_h.at[pl.ds(0, _C)], rows.at[0, 2],
                                  gs.at[0, 2]).wait()
            pltpu.sync_copy(rows.at[0, 2], out_ei.at[pl.ds(off, _C)])
            pltpu.make_async_copy(emb_h.at[pl.ds(0, _C)], rows.at[0, 3],
                                  gs.at[0, 3]).wait()
            pltpu.sync_copy(rows.at[0, 3], out_zi.at[pl.ds(off, _C)])
            pltpu.make_async_copy(dv.at[pl.ds(0, _C)], dbuf.at[0],
                                  gs.at[1, 0]).wait()
            pltpu.sync_copy(dbuf.at[0], out_d.at[pl.ds(off, _C)])
            return carry

        lax.fori_loop(0, _SCPT, chunkk, 0)

    def run_a():
        run(emb_a, y0a, ysa, zsa)
        score(emb_a, zsa, eu_a, zu_a, ei_a, zi_a, du, True)

    def run_b():
        run(emb_b, y0b, ysb, zsb)
        score(emb_b, zsb, eu_b, zu_b, ei_b, zi_b, di, False)

    pl.when(c == 0)(run_a)
    pl.when(c == 1)(run_b)


@functools.lru_cache(maxsize=None)
def _mega_call():
    mesh = plsc.VectorSubcoreMesh(core_axis_name="c", subcore_axis_name="s")
    nh = [jax.ShapeDtypeStruct((NZ, H), _f32)] * 4
    gh = [jax.ShapeDtypeStruct((UP, H), _f32)] * 8
    dh = [jax.ShapeDtypeStruct((UP,), _f32)] * 2
    return pl.kernel(
        _mega_body,
        out_type=nh + gh + dh,
        mesh=mesh,
        compiler_params=pltpu.CompilerParams(use_tc_tiling_on_sc=False,
                                             needs_layout_passes=False),
        scratch_types=[
            pltpu.VMEM((2, _Q, _C), _i32),
            pltpu.VMEM((2, _Q, _C), _i32),
            pltpu.VMEM((2, _Q, _C, H), _f32),
            pltpu.VMEM((2, _C), _f32),
            pltpu.VMEM_SHARED((NZ, H), _f32),
            pltpu.SemaphoreType.DMA((2, _Q)),
            pltpu.SemaphoreType.DMA((2, _Q)),
        ],
    )


def _prep_body(dga, dgb, ea, eb, y0a, y0b, dv1, d21):
    deg = dga[...] + dgb[...]
    d = jnp.where(deg > 0, lax.rsqrt(deg), jnp.zeros_like(deg))
    y0a[...] = ea[...] * d
    y0b[...] = eb[...] * d
    dv1[...] = d
    d21[...] = d * d


def _make_prep():
    G = 16
    R = NZ // G
    n1 = pl.BlockSpec((R, 1), lambda i: (i, 0))
    nh = pl.BlockSpec((R, H), lambda i: (i, 0))
    return pl.pallas_call(
        _prep_body,
        grid=(G,),
        in_specs=[n1, n1, nh, nh],
        out_specs=[nh, nh, n1, n1],
        out_shape=[jax.ShapeDtypeStruct((NZ, H), _f32)] * 2
        + [jax.ShapeDtypeStruct((NZ, 1), _f32)] * 2,
    )


_prep_call = _make_prep()


def _dot_body(eu_a, zu_a, ei_a, zi_a, eu_b, zu_b, ei_b, zi_b, du1, di1, out):
    du = du1[...]
    di = di1[...]
    oua = eu_a[...] + du * zu_a[...]
    oub = eu_b[...] + du * zu_b[...]
    oia = ei_a[...] + di * zi_a[...]
    oib = ei_b[...] + di * zi_b[...]
    out[...] = (ALPHA * ALPHA) * jnp.sum(oua * oia + oub * oib,
                                         axis=1, keepdims=True)


def _make_dot():
    G = 8
    R = UP // G
    n1 = pl.BlockSpec((R, 1), lambda i: (i, 0))
    nh = pl.BlockSpec((R, H), lambda i: (i, 0))
    return pl.pallas_call(
        _dot_body,
        grid=(G,),
        in_specs=[nh] * 8 + [n1, n1],
        out_specs=n1,
        out_shape=jax.ShapeDtypeStruct((UP, 1), _f32),
    )


_dot_call = _make_dot()


def kernel(edge_index, batch, emb):
    src = edge_index[0].astype(_i32)
    dst = edge_index[1].astype(_i32)
    pad = EP - E
    fill = jnp.full((pad,), N, _i32)
    src_p = jnp.concatenate([src, fill])
    dst_p = jnp.concatenate([dst, fill])
    dst2 = dst_p.reshape(EROWS, 128)
    src2c = src_p.reshape(_EC, _C)
    dst2c = dst_p.reshape(_EC, _C)
    embp = jnp.pad(emb, ((0, NZ - N), (0, 0)))
    ea = embp[:, :H]
    eb = embp[:, H:]
    ufill = jnp.full((UP - U,), N, _i32)
    u2 = jnp.concatenate([batch[:, :, 0].reshape(-1).astype(_i32),
                          ufill]).reshape(UP // _C, _C)
    i2 = jnp.concatenate([batch[:, :, 1].reshape(-1).astype(_i32),
                          ufill]).reshape(UP // _C, _C)

    dga, dgb = _deg_call()(dst2)
    dga1 = dga.reshape(NZ, 1)
    dgb1 = dgb.reshape(NZ, 1)
    y0a, y0b, dv1, d21 = _prep_call(dga1, dgb1, ea, eb)

    outs = _mega_call()(src2c, dst2c, ea, eb, y0a, y0b,
                        d21.reshape(NZ), dv1.reshape(NZ), u2, i2)
    (_ysa, _ysb, _zsa, _zsb,
     eu_a, zu_a, ei_a, zi_a, eu_b, zu_b, ei_b, zi_b, du, di) = outs

    logits = _dot_call(eu_a, zu_a, ei_a, zi_a, eu_b, zu_b, ei_b, zi_b,
                       du.reshape(UP, 1), di.reshape(UP, 1))
    return logits[:U].reshape(batch.shape[0], -1)


# speedup vs baseline: 20.5027x; 1.1242x over previous
import functools

import jax
import jax.numpy as jnp
from jax import lax
from jax.experimental import pallas as pl
from jax.experimental.pallas import tpu as pltpu
from jax.experimental.pallas import tpu_sc as plsc

N = 50000
D = 64
H = 32
NLAYERS = 3
ALPHA = 1.0 / (NLAYERS + 1)

NZ = 50048
E = 800000
EP = 802816
EROWS = EP // 128
NT = 16
RPT = NZ // NT
U = 20480
UP = 21504

_C = 112
_Q = 4
_EC = EP // _C
_CPT = _EC // NT
_NG = _CPT // _Q
_SCPT = UP // _C // NT

_f32 = jnp.float32
_i32 = jnp.int32


def _zero_vec128(buf):
    for j in range(8):
        buf[pl.ds(j * 16, 16)] = jnp.zeros((16,), _f32)


def _deg_body(dst2, dga, dgb, idst, ones_v, zbuf, deg_sh, dsem, ssd):
    c = lax.axis_index("c")
    s = lax.axis_index("s")
    for j in range(8):
        ones_v[pl.ds(j * 16, 16)] = jnp.ones((16,), _f32)
    _zero_vec128(zbuf)
    base = s * RPT

    def zcp(j, carry):
        pltpu.sync_copy(zbuf, deg_sh.at[pl.ds(base + j * 128, 128)])
        return carry

    lax.fori_loop(0, RPT // 128, zcp, 0)
    rem = RPT % 128
    if rem:
        pltpu.sync_copy(zbuf.at[pl.ds(0, rem)],
                        deg_sh.at[pl.ds(base + (RPT // 128) * 128, rem)])
    plsc.subcore_barrier()

    nrows = EROWS // 32
    r0 = (c * NT + s) * nrows
    for t in range(3):
        pltpu.async_copy(dst2.at[pl.ds(r0 + t, 1)], idst.at[t], dsem.at[t])

    def body(j, carry):
        p = lax.rem(j, 6)
        q = lax.rem(j + 3, 6)

        def drain3():
            pltpu.make_async_copy(ones_v, deg_sh.at[pl.ds(0, 128)],
                                  ssd.at[q]).wait()

        pl.when(j >= 3)(drain3)

        def preload():
            pltpu.async_copy(dst2.at[pl.ds(r0 + j + 3, 1)], idst.at[q],
                             dsem.at[q])

        pl.when(j + 3 < nrows)(preload)
        pltpu.make_async_copy(dst2.at[pl.ds(r0, 1)], idst.at[p],
                              dsem.at[p]).wait()
        pltpu.async_copy(ones_v, deg_sh.at[idst.at[p, 0]], ssd.at[p],
                         add=True)
        return carry

    lax.fori_loop(0, nrows, body, 0)
    for t in (nrows - 3, nrows - 2, nrows - 1):
        pltpu.make_async_copy(ones_v, deg_sh.at[pl.ds(0, 128)],
                              ssd.at[t % 6]).wait()
    plsc.subcore_barrier()

    def wout(dg):
        def body(j, carry):
            pltpu.sync_copy(deg_sh.at[pl.ds(base + j * 128, 128)], zbuf)
            pltpu.sync_copy(zbuf, dg.at[pl.ds(base + j * 128, 128)])
            return carry

        lax.fori_loop(0, RPT // 128, body, 0)
        if rem:
            off = base + (RPT // 128) * 128
            pltpu.sync_copy(deg_sh.at[pl.ds(off, rem)], zbuf.at[pl.ds(0, rem)])
            pltpu.sync_copy(zbuf.at[pl.ds(0, rem)], dg.at[pl.ds(off, rem)])

    pl.when(c == 0)(lambda: wout(dga))
    pl.when(c == 1)(lambda: wout(dgb))


@functools.lru_cache(maxsize=None)
def _deg_call():
    mesh = plsc.VectorSubcoreMesh(core_axis_name="c", subcore_axis_name="s")
    return pl.kernel(
        _deg_body,
        out_type=[jax.ShapeDtypeStruct((NZ,), _f32)] * 2,
        mesh=mesh,
        compiler_params=pltpu.CompilerParams(use_tc_tiling_on_sc=False),
        scratch_types=[
            pltpu.VMEM((6, 1, 128), _i32),
            pltpu.VMEM((128,), _f32),
            pltpu.VMEM((128,), _f32),
            pltpu.VMEM_SHARED((NZ,), _f32),
            pltpu.SemaphoreType.DMA((6,)),
            pltpu.SemaphoreType.DMA((6,)),
        ],
    )


def _mega_body(src2, dst2, emb_a, emb_b, y0a, y0b, d2, dv, u2, i2,
               ysa, ysb, zsa, zsb,
               eu_a, zu_a, ei_a, zi_a, eu_b, zu_b, ei_b, zi_b, du, di,
               isrc, idst, rows, dbuf, z_sh, gs, ss):
    c = lax.axis_index("c")
    s = lax.axis_index("s")
    base = s * RPT
    nzc = RPT // _C
    rem = RPT % _C

    def zero_rows00():
        def zrow(i, carry):
            rows[0, 0, i, pl.ds(0, 16)] = jnp.zeros((16,), _f32)
            rows[0, 0, i, pl.ds(16, 16)] = jnp.zeros((16,), _f32)
            return carry

        lax.fori_loop(0, _C, zrow, 0)

    def zero_stripe():
        def zcp(j, carry):
            pltpu.sync_copy(rows.at[0, 0], z_sh.at[pl.ds(base + j * _C, _C)])
            return carry

        lax.fori_loop(0, nzc, zcp, 0)
        if rem:
            pltpu.sync_copy(rows.at[0, 0].at[pl.ds(0, rem)],
                            z_sh.at[pl.ds(base + nzc * _C, rem)])

    def scatter_layer(tab):
        r0 = s * _CPT

        def load_idx(g, b):
            pltpu.sync_copy(src2.at[pl.ds(r0 + g * _Q, _Q)], isrc.at[b])
            pltpu.sync_copy(dst2.at[pl.ds(r0 + g * _Q, _Q)], idst.at[b])

        def issue_group(b):
            for k in range(_Q):
                pltpu.async_copy(tab.at[isrc.at[b, k]],
                                 rows.at[b, k], gs.at[b, k])

        def wait_scatters(b):
            for k in range(_Q):
                pltpu.make_async_copy(rows.at[b, k],
                                      z_sh.at[pl.ds(0, _C)],
                                      ss.at[b, k]).wait()

        def drain_and_scatter(b):
            for k in range(_Q):
                pltpu.make_async_copy(tab.at[pl.ds(0, _C)],
                                      rows.at[b, k], gs.at[b, k]).wait()
                pltpu.async_copy(rows.at[b, k],
                                 z_sh.at[idst.at[b, k]], ss.at[b, k],
                                 add=True)

        load_idx(0, 0)
        issue_group(0)

        def body(g, carry):
            b = lax.rem(g, 2)
            nb = 1 - b

            def advance():
                pltpu.async_copy(src2.at[pl.ds(r0 + (g + 1) * _Q, _Q)],
                                 isrc.at[nb], gs.at[nb, 0])
                pl.when(g >= 1)(lambda: wait_scatters(nb))
                pltpu.async_copy(dst2.at[pl.ds(r0 + (g + 1) * _Q, _Q)],
                                 idst.at[nb], gs.at[nb, 1])
                pltpu.make_async_copy(src2.at[pl.ds(r0, _Q)],
                                      isrc.at[nb], gs.at[nb, 0]).wait()
                pltpu.make_async_copy(dst2.at[pl.ds(r0, _Q)],
                                      idst.at[nb], gs.at[nb, 1]).wait()
                issue_group(nb)

            pl.when(g + 1 < _NG)(advance)
            drain_and_scatter(b)
            return carry

        lax.fori_loop(0, _NG, body, 0)
        wait_scatters((_NG - 2) % 2)
        wait_scatters((_NG - 1) % 2)

    def scale_rows(nr, zslot, yslot, dslot):
        def srow(r, carry):
            dvec = plsc.load_gather(dslot, [jnp.full((16,), r, _i32)])
            yslot[r, pl.ds(0, 16)] = zslot[r, pl.ds(0, 16)] * dvec
            yslot[r, pl.ds(16, 16)] = zslot[r, pl.ds(16, 16)] * dvec
            return carry

        lax.fori_loop(0, nr, srow, 0)

    def add_rows(nr, dst_slot, src_slot):
        def arow(r, carry):
            dst_slot[r, pl.ds(0, 16)] = (dst_slot[r, pl.ds(0, 16)]
                                         + src_slot[r, pl.ds(0, 16)])
            dst_slot[r, pl.ds(16, 16)] = (dst_slot[r, pl.ds(16, 16)]
                                          + src_slot[r, pl.ds(16, 16)])
            return carry

        lax.fori_loop(0, nr, arow, 0)

    def writeout(layer, ys, zs):
        def issue_loads(j, b):
            off = base + j * _C
            pltpu.async_copy(z_sh.at[pl.ds(off, _C)], rows.at[b, 0],
                             gs.at[b, 0])
            if layer < NLAYERS:
                pltpu.async_copy(d2.at[pl.ds(off, _C)], dbuf.at[b],
                                 gs.at[b, 1])
            if layer > 1:
                pltpu.async_copy(zs.at[pl.ds(off, _C)], rows.at[b, 2],
                                 gs.at[b, 2])

        def wait_loads(b):
            pltpu.make_async_copy(z_sh.at[pl.ds(base, _C)], rows.at[b, 0],
                                  gs.at[b, 0]).wait()
            if layer < NLAYERS:
                pltpu.make_async_copy(d2.at[pl.ds(base, _C)], dbuf.at[b],
                                      gs.at[b, 1]).wait()
            if layer > 1:
                pltpu.make_async_copy(zs.at[pl.ds(base, _C)], rows.at[b, 2],
                                      gs.at[b, 2]).wait()

        def wait_stores(b):
            if layer < NLAYERS:
                pltpu.make_async_copy(rows.at[b, 1], ys.at[pl.ds(base, _C)],
                                      gs.at[b, 3]).wait()
            pltpu.make_async_copy(rows.at[b, 2], zs.at[pl.ds(base, _C)],
                                  gs.at[b, 3]).wait()

        def compute_and_store(j, b):
            off = base + j * _C
            if layer < NLAYERS:
                scale_rows(_C, rows.at[b, 0], rows.at[b, 1], dbuf.at[b])
                pltpu.async_copy(rows.at[b, 1], ys.at[pl.ds(off, _C)],
                                 gs.at[b, 3])
            if layer > 1:
                add_rows(_C, rows.at[b, 2], rows.at[b, 0])
                pltpu.async_copy(rows.at[b, 2], zs.at[pl.ds(off, _C)],
                                 gs.at[b, 3])
            else:
                pltpu.async_copy(rows.at[b, 0], zs.at[pl.ds(off, _C)],
                                 gs.at[b, 3])

        issue_loads(0, 0)

        def wb(j, carry):
            b = lax.rem(j, 2)
            nb = 1 - b

            def advance():
                pl.when(j >= 1)(lambda: wait_stores(nb))
                issue_loads(j + 1, nb)

            pl.when(j + 1 < nzc)(advance)
            wait_loads(b)
            compute_and_store(j, b)
            return carry

        lax.fori_loop(0, nzc, wb, 0)
        wait_stores((nzc - 2) % 2)
        wait_stores((nzc - 1) % 2)

        if rem:
            off = base + nzc * _C
            zsl = rows.at[0, 0].at[pl.ds(0, rem)]
            ysl = rows.at[0, 1].at[pl.ds(0, rem)]
            ssl = rows.at[0, 2].at[pl.ds(0, rem)]
            pltpu.sync_copy(z_sh.at[pl.ds(off, rem)], zsl)
            if layer < NLAYERS:
                pltpu.sync_copy(d2.at[pl.ds(off, rem)],
                                dbuf.at[0].at[pl.ds(0, rem)])
                scale_rows(rem, rows.at[0, 0], rows.at[0, 1], dbuf.at[0])
                pltpu.sync_copy(ysl, ys.at[pl.ds(off, rem)])
            if layer == 1:
                pltpu.sync_copy(zsl, zs.at[pl.ds(off, rem)])
            else:
                pltpu.sync_copy(zs.at[pl.ds(off, rem)], ssl)
                add_rows(rem, rows.at[0, 2], rows.at[0, 0])
                pltpu.sync_copy(ssl, zs.at[pl.ds(off, rem)])

    def run(emb_h, y0_h, ys, zs):
        for layer in (1, 2, 3):
            zero_rows00()
            zero_stripe()
            plsc.subcore_barrier()
            scatter_layer(y0_h if layer == 1 else ys)
            plsc.subcore_barrier()
            writeout(layer, ys, zs)
            plsc.subcore_barrier()

    def score(emb_h, zs, out_eu, out_zu, out_ei, out_zi, out_d, d_from_u):
        def chunkk(m, carry):
            r = s * _SCPT + m
            off = r * _C
            uslot = isrc.at[0, 0]
            islot = isrc.at[0, 1]
            pltpu.sync_copy(u2.at[r], uslot)
            pltpu.sync_copy(i2.at[r], islot)
            pltpu.async_copy(emb_h.at[uslot], rows.at[0, 0], gs.at[0, 0])
            pltpu.async_copy(zs.at[uslot], rows.at[0, 1], gs.at[0, 1])
            pltpu.async_copy(emb_h.at[islot], rows.at[0, 2], gs.at[0, 2])
            pltpu.async_copy(zs.at[islot], rows.at[0, 3], gs.at[0, 3])
            dslot = uslot if d_from_u else islot
            pltpu.async_copy(dv.at[dslot], dbuf.at[0], gs.at[1, 0])
            pltpu.make_async_copy(emb_h.at[pl.ds(0, _C)], rows.at[0, 0],
                                  gs.at[0, 0]).wait()
            pltpu.sync_copy(rows.at[0, 0], out_eu.at[pl.ds(off, _C)])
            pltpu.make_async_copy(emb_h.at[pl.ds(0, _C)], rows.at[0, 1],
                                  gs.at[0, 1]).wait()
            pltpu.sync_copy(rows.at[0, 1], out_zu.at[pl.ds(off, _C)])
            pltpu.make_async_copy(emb_h.at[pl.ds(0, _C)], rows.at[0, 2],
                                  gs.at[0, 2]).wait()
            pltpu.sync_copy(rows.at[0, 2], out_ei.at[pl.ds(off, _C)])
            pltpu.make_async_copy(emb_h.at[pl.ds(0, _C)], rows.at[0, 3],
                                  gs.at[0, 3]).wait()
            pltpu.sync_copy(rows.at[0, 3], out_zi.at[pl.ds(off, _C)])
            pltpu.make_async_copy(dv.at[pl.ds(0, _C)], dbuf.at[0],
                                  gs.at[1, 0]).wait()
            pltpu.sync_copy(dbuf.at[0], out_d.at[pl.ds(off, _C)])
            return carry

        lax.fori_loop(0, _SCPT, chunkk, 0)

    def run_a():
        run(emb_a, y0a, ysa, zsa)
        score(emb_a, zsa, eu_a, zu_a, ei_a, zi_a, du, True)

    def run_b():
        run(emb_b, y0b, ysb, zsb)
        score(emb_b, zsb, eu_b, zu_b, ei_b, zi_b, di, False)

    pl.when(c == 0)(run_a)
    pl.when(c == 1)(run_b)


@functools.lru_cache(maxsize=None)
def _mega_call():
    mesh = plsc.VectorSubcoreMesh(core_axis_name="c", subcore_axis_name="s")
    nh = [jax.ShapeDtypeStruct((NZ, H), _f32)] * 4
    gh = [jax.ShapeDtypeStruct((UP, H), _f32)] * 8
    dh = [jax.ShapeDtypeStruct((UP,), _f32)] * 2
    return pl.kernel(
        _mega_body,
        out_type=nh + gh + dh,
        mesh=mesh,
        compiler_params=pltpu.CompilerParams(use_tc_tiling_on_sc=False,
                                             needs_layout_passes=False),
        scratch_types=[
            pltpu.VMEM((2, _Q, _C), _i32),
            pltpu.VMEM((2, _Q, _C), _i32),
            pltpu.VMEM((2, _Q, _C, H), _f32),
            pltpu.VMEM((2, _C), _f32),
            pltpu.VMEM_SHARED((NZ, H), _f32),
            pltpu.SemaphoreType.DMA((2, _Q)),
            pltpu.SemaphoreType.DMA((2, _Q)),
        ],
    )


def _prep_body(dga, dgb, ea, eb, y0a, y0b, dv1, d21):
    deg = dga[...] + dgb[...]
    d = jnp.where(deg > 0, lax.rsqrt(deg), jnp.zeros_like(deg))
    y0a[...] = ea[...] * d
    y0b[...] = eb[...] * d
    dv1[...] = d
    d21[...] = d * d


def _make_prep():
    G = 16
    R = NZ // G
    n1 = pl.BlockSpec((R, 1), lambda i: (i, 0))
    nh = pl.BlockSpec((R, H), lambda i: (i, 0))
    return pl.pallas_call(
        _prep_body,
        grid=(G,),
        in_specs=[n1, n1, nh, nh],
        out_specs=[nh, nh, n1, n1],
        out_shape=[jax.ShapeDtypeStruct((NZ, H), _f32)] * 2
        + [jax.ShapeDtypeStruct((NZ, 1), _f32)] * 2,
    )


_prep_call = _make_prep()


def _dot_body(eu_a, zu_a, ei_a, zi_a, eu_b, zu_b, ei_b, zi_b, du1, di1, out):
    du = du1[...]
    di = di1[...]
    oua = eu_a[...] + du * zu_a[...]
    oub = eu_b[...] + du * zu_b[...]
    oia = ei_a[...] + di * zi_a[...]
    oib = ei_b[...] + di * zi_b[...]
    out[...] = (ALPHA * ALPHA) * jnp.sum(oua * oia + oub * oib,
                                         axis=1, keepdims=True)


def _make_dot():
    G = 8
    R = UP // G
    n1 = pl.BlockSpec((R, 1), lambda i: (i, 0))
    nh = pl.BlockSpec((R, H), lambda i: (i, 0))
    return pl.pallas_call(
        _dot_body,
        grid=(G,),
        in_specs=[nh] * 8 + [n1, n1],
        out_specs=n1,
        out_shape=jax.ShapeDtypeStruct((UP, 1), _f32),
    )


_dot_call = _make_dot()


def kernel(edge_index, batch, emb):
    src = edge_index[0].astype(_i32)
    dst = edge_index[1].astype(_i32)
    pad = EP - E
    fill = jnp.full((pad,), N, _i32)
    src_p = jnp.concatenate([src, fill])
    dst_p = jnp.concatenate([dst, fill])
    dst2 = dst_p.reshape(EROWS, 128)
    src2c = src_p.reshape(_EC, _C)
    dst2c = dst_p.reshape(_EC, _C)
    embp = jnp.pad(emb, ((0, NZ - N), (0, 0)))
    ea = embp[:, :H]
    eb = embp[:, H:]
    ufill = jnp.full((UP - U,), N, _i32)
    u2 = jnp.concatenate([batch[:, :, 0].reshape(-1).astype(_i32),
                          ufill]).reshape(UP // _C, _C)
    i2 = jnp.concatenate([batch[:, :, 1].reshape(-1).astype(_i32),
                          ufill]).reshape(UP // _C, _C)

    dga, dgb = _deg_call()(dst2)
    dga1 = dga.reshape(NZ, 1)
    dgb1 = dgb.reshape(NZ, 1)
    y0a, y0b, dv1, d21 = _prep_call(dga1, dgb1, ea, eb)

    outs = _mega_call()(src2c, dst2c, ea, eb, y0a, y0b,
                        d21.reshape(NZ), dv1.reshape(NZ), u2, i2)
    (_ysa, _ysb, _zsa, _zsb,
     eu_a, zu_a, ei_a, zi_a, eu_b, zu_b, ei_b, zi_b, du, di) = outs

    logits = _dot_call(eu_a, zu_a, ei_a, zi_a, eu_b, zu_b, ei_b, zi_b,
                       du.reshape(UP, 1), di.reshape(UP, 1))
    return logits[:U].reshape(batch.shape[0], -1)
